# Initial kernel scaffold; baseline (speedup 1.0000x reference)
import jax
import jax.numpy as jnp
from jax.experimental import pallas as pl


def kernel(V, E, edge_index, rev_edge_index, batch, weight, W_i, W_h, W_o, b_o):
    raise NotImplementedError("write your pallas kernel here")


# speedup vs baseline: 1.6066x; 1.6066x over previous
import functools

import jax
import jax.numpy as jnp
from jax import lax
from jax.experimental import pallas as pl
from jax.experimental.pallas import tpu as pltpu
from jax.experimental.pallas import tpu_sc as plsc

NC = 2
NS = 16
NW = NC * NS
LANES = 16

HIDDEN = 128


def _sc_mesh():
    return plsc.VectorSubcoreMesh(core_axis_name="c", subcore_axis_name="s")


def _make_gather(B, D, CH):
    b_per_w = B // NW
    n_ch = b_per_w // CH
    assert b_per_w % CH == 0 and B % (8 * NW) == 0 and CH % 8 == 0

    @functools.partial(
        pl.kernel,
        mesh=_sc_mesh(),
        out_type=jax.ShapeDtypeStruct((B, D), jnp.float32),
        scratch_types=[
            pltpu.VMEM((CH,), jnp.int32),
            pltpu.VMEM((CH, D), jnp.float32),
            pltpu.SemaphoreType.DMA,
        ],
    )
    def gather_k(table_hbm, idx_hbm, out_hbm, idx_v, rows_v, sem):
        wid = lax.axis_index("s") * NC + lax.axis_index("c")
        base = wid * b_per_w

        def body(c, carry):
            off = base + c * CH
            pltpu.sync_copy(idx_hbm.at[pl.ds(off, CH)], idx_v)
            pltpu.async_copy(table_hbm.at[idx_v], rows_v, sem).wait()
            pltpu.sync_copy(rows_v, out_hbm.at[pl.ds(off, CH)])
            return carry

        lax.fori_loop(0, n_ch, body, 0)

    return gather_k


def _make_scatter_add(B, T, D, CH):
    b_per_w = B // NW
    n_ch = b_per_w // CH
    rpt = T // NS
    assert b_per_w % CH == 0 and T % NS == 0 and CH % 8 == 0 and rpt % 8 == 0

    @functools.partial(
        pl.kernel,
        mesh=_sc_mesh(),
        out_type=jax.ShapeDtypeStruct((NC, T, D), jnp.float32),
        scratch_types=[
            pltpu.VMEM((CH,), jnp.int32),
            pltpu.VMEM((CH, D), jnp.float32),
            pltpu.VMEM_SHARED((T, D), jnp.float32),
        ],
    )
    def scatter_k(vals_hbm, dest_hbm, out_hbm, idx_v, rows_v, acc_sh):
        cid = lax.axis_index("c")
        sid = lax.axis_index("s")
        wid = sid * NC + cid
        base = wid * b_per_w

        zero = jnp.zeros((LANES,), jnp.float32)
        zch = CH if rpt % CH == 0 else rpt
        assert rpt % zch == 0 and zch <= CH

        def zrow(r, carry):
            for j in range(D // LANES):
                rows_v[r, pl.ds(j * LANES, LANES)] = zero
            return carry

        lax.fori_loop(0, zch, zrow, 0)

        def zcopy(k, carry):
            pltpu.sync_copy(
                rows_v.at[pl.ds(0, zch)],
                acc_sh.at[pl.ds(sid * rpt + k * zch, zch)],
            )
            return carry

        lax.fori_loop(0, rpt // zch, zcopy, 0)
        plsc.subcore_barrier()

        def body(c, carry):
            off = base + c * CH
            pltpu.sync_copy(vals_hbm.at[pl.ds(off, CH)], rows_v)
            pltpu.sync_copy(dest_hbm.at[pl.ds(off, CH)], idx_v)
            pltpu.sync_copy(rows_v, acc_sh.at[idx_v], add=True)
            return carry

        lax.fori_loop(0, n_ch, body, 0)
        plsc.subcore_barrier()
        pltpu.sync_copy(
            acc_sh.at[pl.ds(sid * rpt, rpt)],
            out_hbm.at[cid, pl.ds(sid * rpt, rpt)],
        )

    return scatter_k


def _proj_body(x_ref, w_ref, o_ref):
    o_ref[...] = jnp.dot(x_ref[...], w_ref[...],
                         preferred_element_type=jnp.float32)


def _init_body(asrc_ref, e_ref, wb_ref, w_ref, h0_ref, hw_ref):
    h0 = jnp.maximum(
        asrc_ref[...]
        + jnp.dot(e_ref[...], wb_ref[...], preferred_element_type=jnp.float32),
        0.0,
    )
    h0_ref[...] = h0
    hw_ref[...] = h0 * w_ref[...]


def _addp_body(p_ref, o_ref):
    o_ref[...] = p_ref[0] + p_ref[1]


def _combine_body(msrc_ref, hwrev_ref, h0_ref, w_ref, wh_ref, h_ref, hw_ref):
    m = msrc_ref[...] - hwrev_ref[...]
    h = jnp.maximum(
        jnp.dot(m, wh_ref[...], preferred_element_type=jnp.float32)
        + h0_ref[...],
        0.0,
    )
    h_ref[...] = h
    hw_ref[...] = h * w_ref[...]


def _atom_body(v_ref, mf_ref, woa_ref, wob_ref, b_ref, o_ref):
    m = mf_ref[0] + mf_ref[1]
    o_ref[...] = jnp.maximum(
        jnp.dot(v_ref[...], woa_ref[...], preferred_element_type=jnp.float32)
        + jnp.dot(m, wob_ref[...], preferred_element_type=jnp.float32)
        + b_ref[...],
        0.0,
    )


def _row_spec(br, d):
    return pl.BlockSpec((br, d), lambda i: (i, 0))


def _full_spec(shape):
    return pl.BlockSpec(shape, lambda i: tuple(0 for _ in shape))


def kernel(V, E, edge_index, rev_edge_index, batch, weight, W_i, W_h, W_o, b_o):
    n_atoms, atom_dim = V.shape
    n_edges, bond_dim = E.shape
    hidden = W_h.shape[0]

    src = edge_index[0]
    dest = edge_index[1]
    w2 = weight[:, None]
    wa = W_i[:atom_dim]
    wb = W_i[atom_dim:]
    woa = W_o[:atom_dim]
    wob = W_o[atom_dim:]
    b2 = b_o[None, :]

    BR_E = 2000
    BR_A = 2000
    CH = 80

    t_acc = 10240
    gather = _make_gather(n_edges, hidden, CH)
    scatter_edges = _make_scatter_add(n_edges, t_acc, hidden, CH)

    A = pl.pallas_call(
        _proj_body,
        grid=(n_atoms // BR_A,),
        in_specs=[_row_spec(BR_A, atom_dim), _full_spec((atom_dim, hidden))],
        out_specs=_row_spec(BR_A, hidden),
        out_shape=jax.ShapeDtypeStruct((n_atoms, hidden), jnp.float32),
    )(V, wa)

    a_src = gather(A, src)

    h0, hw = pl.pallas_call(
        _init_body,
        grid=(n_edges // BR_E,),
        in_specs=[
            _row_spec(BR_E, hidden),
            _row_spec(BR_E, bond_dim),
            _full_spec((bond_dim, hidden)),
            _row_spec(BR_E, 1),
        ],
        out_specs=[_row_spec(BR_E, hidden), _row_spec(BR_E, hidden)],
        out_shape=[
            jax.ShapeDtypeStruct((n_edges, hidden), jnp.float32),
            jax.ShapeDtypeStruct((n_edges, hidden), jnp.float32),
        ],
    )(a_src, E, wb, w2)

    add_partials = pl.pallas_call(
        _addp_body,
        grid=(t_acc // 2048,),
        in_specs=[pl.BlockSpec((NC, 2048, hidden), lambda i: (0, i, 0))],
        out_specs=_row_spec(2048, hidden),
        out_shape=jax.ShapeDtypeStruct((t_acc, hidden), jnp.float32),
    )

    combine = pl.pallas_call(
        _combine_body,
        grid=(n_edges // BR_E,),
        in_specs=[
            _row_spec(BR_E, hidden),
            _row_spec(BR_E, hidden),
            _row_spec(BR_E, hidden),
            _row_spec(BR_E, 1),
            _full_spec((hidden, hidden)),
        ],
        out_specs=[_row_spec(BR_E, hidden), _row_spec(BR_E, hidden)],
        out_shape=[
            jax.ShapeDtypeStruct((n_edges, hidden), jnp.float32),
            jax.ShapeDtypeStruct((n_edges, hidden), jnp.float32),
        ],
    )

    h = h0
    for _ in range(3):
        s = add_partials(scatter_edges(hw, dest))
        m_src = gather(s, src)
        hw_rev = gather(hw, rev_edge_index)
        h, hw = combine(m_src, hw_rev, h0, w2, W_h)

    mf = scatter_edges(h, dest)

    h_atom = pl.pallas_call(
        _atom_body,
        grid=(n_atoms // BR_A,),
        in_specs=[
            _row_spec(BR_A, atom_dim),
            pl.BlockSpec((NC, BR_A, hidden), lambda i: (0, i, 0)),
            _full_spec((atom_dim, hidden)),
            _full_spec((hidden, hidden)),
            pl.BlockSpec((1, hidden), lambda i: (0, 0)),
        ],
        out_specs=_row_spec(BR_A, hidden),
        out_shape=jax.ShapeDtypeStruct((n_atoms, hidden), jnp.float32),
    )(V, mf, woa, wob, b2)

    n_pad = 10240
    n_mols = 256
    hp = jnp.pad(h_atom, ((0, n_pad - n_atoms), (0, 0)))
    bp = jnp.pad(batch, (0, n_pad - n_atoms))
    scatter_pool = _make_scatter_add(n_pad, n_mols, hidden, CH)
    pm = scatter_pool(hp, bp)
    mol_vecs = pl.pallas_call(
        _addp_body,
        grid=(1,),
        in_specs=[pl.BlockSpec((NC, n_mols, hidden), lambda i: (0, 0, 0))],
        out_specs=pl.BlockSpec((n_mols, hidden), lambda i: (0, 0)),
        out_shape=jax.ShapeDtypeStruct((n_mols, hidden), jnp.float32),
    )(pm)

    return (h_atom, batch, mol_vecs, h)


# speedup vs baseline: 2.4907x; 1.5503x over previous
import functools

import jax
import jax.numpy as jnp
from jax import lax
from jax.experimental import pallas as pl
from jax.experimental.pallas import tpu as pltpu
from jax.experimental.pallas import tpu_sc as plsc

NC = 2
NS = 16
NW = NC * NS
LANES = 16

HIDDEN = 128


def _sc_mesh():
    return plsc.VectorSubcoreMesh(core_axis_name="c", subcore_axis_name="s")


def _make_gather(B, D, CH, NBUF):
    b_per_w = B // NW
    n_ch = b_per_w // CH
    n_grp = n_ch // NBUF
    assert b_per_w % CH == 0 and B % (8 * NW) == 0 and CH % 8 == 0
    assert n_ch % NBUF == 0 and CH <= 128

    @functools.partial(
        pl.kernel,
        mesh=_sc_mesh(),
        out_type=jax.ShapeDtypeStruct((B, D), jnp.float32),
        scratch_types=[
            *[pltpu.VMEM((CH,), jnp.int32) for _ in range(NBUF)],
            *[pltpu.VMEM((CH, D), jnp.float32) for _ in range(NBUF)],
            *[pltpu.SemaphoreType.DMA for _ in range(3 * NBUF)],
        ],
    )
    def gather_k(table_hbm, idx_hbm, out_hbm, *rest):
        ibufs = rest[:NBUF]
        bufs = rest[NBUF:2 * NBUF]
        isem = rest[2 * NBUF:3 * NBUF]
        gsem = rest[3 * NBUF:4 * NBUF]
        osem = rest[4 * NBUF:]
        wid = lax.axis_index("s") * NC + lax.axis_index("c")
        base = wid * b_per_w

        def start_idx(c, b):
            pltpu.async_copy(idx_hbm.at[pl.ds(base + c * CH, CH)], ibufs[b],
                             isem[b])

        def wait_idx(b):
            pltpu.make_async_copy(idx_hbm.at[pl.ds(base, CH)], ibufs[b],
                                  isem[b]).wait()

        def start_gather(b):
            pltpu.async_copy(table_hbm.at[ibufs[b]], bufs[b], gsem[b])

        def wait_gather(b):
            pltpu.make_async_copy(table_hbm.at[ibufs[b]], bufs[b],
                                  gsem[b]).wait()

        def start_out(c, b):
            pltpu.async_copy(bufs[b], out_hbm.at[pl.ds(base + c * CH, CH)],
                             osem[b])

        def wait_out(b):
            pltpu.make_async_copy(bufs[b], out_hbm.at[pl.ds(base, CH)],
                                  osem[b]).wait()

        for b in range(NBUF):
            start_idx(b, b)

        def body(g, carry):
            c0 = g * NBUF
            for b in range(NBUF):
                wait_idx(b)

                @pl.when(g > 0)
                def _():
                    wait_out(b)

                start_gather(b)
            for b in range(NBUF):
                wait_gather(b)
                start_out(c0 + b, b)

                @pl.when(g < n_grp - 1)
                def _():
                    start_idx(c0 + NBUF + b, b)

            return carry

        lax.fori_loop(0, n_grp, body, 0)
        for b in range(NBUF):
            wait_out(b)

    return gather_k


def _make_scatter_add(B, T, D, CH, NBUF):
    b_per_w = B // NW
    n_ch = b_per_w // CH
    n_grp = n_ch // NBUF
    rpt = T // NS
    assert b_per_w % CH == 0 and T % NS == 0 and CH % 8 == 0 and rpt % 8 == 0
    assert n_ch % NBUF == 0 and CH <= 128

    @functools.partial(
        pl.kernel,
        mesh=_sc_mesh(),
        out_type=jax.ShapeDtypeStruct((NC, T, D), jnp.float32),
        scratch_types=[
            *[pltpu.VMEM((CH,), jnp.int32) for _ in range(NBUF)],
            *[pltpu.VMEM((CH, D), jnp.float32) for _ in range(NBUF)],
            pltpu.VMEM_SHARED((T, D), jnp.float32),
            *[pltpu.SemaphoreType.DMA for _ in range(2 * NBUF)],
        ],
    )
    def scatter_k(vals_hbm, dest_hbm, out_hbm, *rest):
        ibufs = rest[:NBUF]
        bufs = rest[NBUF:2 * NBUF]
        acc_sh = rest[2 * NBUF]
        lsem = rest[2 * NBUF + 1:3 * NBUF + 1]
        asem = rest[3 * NBUF + 1:]
        cid = lax.axis_index("c")
        sid = lax.axis_index("s")
        wid = sid * NC + cid
        base = wid * b_per_w

        zero = jnp.zeros((LANES,), jnp.float32)
        zch = CH if rpt % CH == 0 else rpt
        assert rpt % zch == 0 and zch <= CH

        def zrow(r, carry):
            for j in range(D // LANES):
                bufs[0][r, pl.ds(j * LANES, LANES)] = zero
            return carry

        lax.fori_loop(0, zch, zrow, 0)

        def zcopy(k, carry):
            pltpu.sync_copy(
                bufs[0].at[pl.ds(0, zch)],
                acc_sh.at[pl.ds(sid * rpt + k * zch, zch)],
            )
            return carry

        lax.fori_loop(0, rpt // zch, zcopy, 0)
        plsc.subcore_barrier()

        def start_load(c, b):
            pltpu.async_copy(vals_hbm.at[pl.ds(base + c * CH, CH)], bufs[b],
                             lsem[b])
            pltpu.async_copy(dest_hbm.at[pl.ds(base + c * CH, CH)], ibufs[b],
                             lsem[b])

        def wait_load(b):
            pltpu.make_async_copy(vals_hbm.at[pl.ds(base, CH)], bufs[b],
                                  lsem[b]).wait()
            pltpu.make_async_copy(dest_hbm.at[pl.ds(base, CH)], ibufs[b],
                                  lsem[b]).wait()

        def start_add(b):
            pltpu.async_copy(bufs[b], acc_sh.at[ibufs[b]], asem[b], add=True)

        def wait_add(b):
            pltpu.make_async_copy(bufs[b], acc_sh.at[ibufs[b]],
                                  asem[b]).wait()

        for b in range(NBUF):
            start_load(b, b)

        def body(g, carry):
            c0 = g * NBUF
            for b in range(NBUF):
                wait_load(b)
                start_add(b)
            for b in range(NBUF):
                wait_add(b)

                @pl.when(g < n_grp - 1)
                def _():
                    start_load(c0 + NBUF + b, b)

            return carry

        lax.fori_loop(0, n_grp, body, 0)
        plsc.subcore_barrier()
        pltpu.sync_copy(
            acc_sh.at[pl.ds(sid * rpt, rpt)],
            out_hbm.at[cid, pl.ds(sid * rpt, rpt)],
        )

    return scatter_k


def _proj_body(x_ref, w_ref, o_ref):
    o_ref[...] = jnp.dot(x_ref[...], w_ref[...],
                         preferred_element_type=jnp.float32)


def _init_body(asrc_ref, e_ref, wb_ref, w_ref, h0_ref, hw_ref):
    h0 = jnp.maximum(
        asrc_ref[...]
        + jnp.dot(e_ref[...], wb_ref[...], preferred_element_type=jnp.float32),
        0.0,
    )
    h0_ref[...] = h0
    hw_ref[...] = h0 * w_ref[...]


def _addp_body(p_ref, o_ref):
    o_ref[...] = p_ref[0] + p_ref[1]


def _combine_body(msrc_ref, hwrev_ref, h0_ref, w_ref, wh_ref, h_ref, hw_ref):
    m = msrc_ref[...] - hwrev_ref[...]
    h = jnp.maximum(
        jnp.dot(m, wh_ref[...], preferred_element_type=jnp.float32)
        + h0_ref[...],
        0.0,
    )
    h_ref[...] = h
    hw_ref[...] = h * w_ref[...]


def _atom_body(v_ref, mf_ref, woa_ref, wob_ref, b_ref, o_ref):
    m = mf_ref[0] + mf_ref[1]
    o_ref[...] = jnp.maximum(
        jnp.dot(v_ref[...], woa_ref[...], preferred_element_type=jnp.float32)
        + jnp.dot(m, wob_ref[...], preferred_element_type=jnp.float32)
        + b_ref[...],
        0.0,
    )


def _row_spec(br, d):
    return pl.BlockSpec((br, d), lambda i: (i, 0))


def _full_spec(shape):
    return pl.BlockSpec(shape, lambda i: tuple(0 for _ in shape))


def kernel(V, E, edge_index, rev_edge_index, batch, weight, W_i, W_h, W_o, b_o):
    n_atoms, atom_dim = V.shape
    n_edges, bond_dim = E.shape
    hidden = W_h.shape[0]

    src = edge_index[0]
    dest = edge_index[1]
    w2 = weight[:, None]
    wa = W_i[:atom_dim]
    wb = W_i[atom_dim:]
    woa = W_o[:atom_dim]
    wob = W_o[atom_dim:]
    b2 = b_o[None, :]

    BR_E = 2000
    BR_A = 2000
    CH = 40
    NBUF = 5

    t_acc = 10240
    gather = _make_gather(n_edges, hidden, CH, NBUF)
    scatter_edges = _make_scatter_add(n_edges, t_acc, hidden, CH, NBUF)

    A = pl.pallas_call(
        _proj_body,
        grid=(n_atoms // BR_A,),
        in_specs=[_row_spec(BR_A, atom_dim), _full_spec((atom_dim, hidden))],
        out_specs=_row_spec(BR_A, hidden),
        out_shape=jax.ShapeDtypeStruct((n_atoms, hidden), jnp.float32),
    )(V, wa)

    a_src = gather(A, src)

    h0, hw = pl.pallas_call(
        _init_body,
        grid=(n_edges // BR_E,),
        in_specs=[
            _row_spec(BR_E, hidden),
            _row_spec(BR_E, bond_dim),
            _full_spec((bond_dim, hidden)),
            _row_spec(BR_E, 1),
        ],
        out_specs=[_row_spec(BR_E, hidden), _row_spec(BR_E, hidden)],
        out_shape=[
            jax.ShapeDtypeStruct((n_edges, hidden), jnp.float32),
            jax.ShapeDtypeStruct((n_edges, hidden), jnp.float32),
        ],
    )(a_src, E, wb, w2)

    add_partials = pl.pallas_call(
        _addp_body,
        grid=(t_acc // 2048,),
        in_specs=[pl.BlockSpec((NC, 2048, hidden), lambda i: (0, i, 0))],
        out_specs=_row_spec(2048, hidden),
        out_shape=jax.ShapeDtypeStruct((t_acc, hidden), jnp.float32),
    )

    combine = pl.pallas_call(
        _combine_body,
        grid=(n_edges // BR_E,),
        in_specs=[
            _row_spec(BR_E, hidden),
            _row_spec(BR_E, hidden),
            _row_spec(BR_E, hidden),
            _row_spec(BR_E, 1),
            _full_spec((hidden, hidden)),
        ],
        out_specs=[_row_spec(BR_E, hidden), _row_spec(BR_E, hidden)],
        out_shape=[
            jax.ShapeDtypeStruct((n_edges, hidden), jnp.float32),
            jax.ShapeDtypeStruct((n_edges, hidden), jnp.float32),
        ],
    )

    h = h0
    for _ in range(3):
        s = add_partials(scatter_edges(hw, dest))
        m_src = gather(s, src)
        hw_rev = gather(hw, rev_edge_index)
        h, hw = combine(m_src, hw_rev, h0, w2, W_h)

    mf = scatter_edges(h, dest)

    h_atom = pl.pallas_call(
        _atom_body,
        grid=(n_atoms // BR_A,),
        in_specs=[
            _row_spec(BR_A, atom_dim),
            pl.BlockSpec((NC, BR_A, hidden), lambda i: (0, i, 0)),
            _full_spec((atom_dim, hidden)),
            _full_spec((hidden, hidden)),
            pl.BlockSpec((1, hidden), lambda i: (0, 0)),
        ],
        out_specs=_row_spec(BR_A, hidden),
        out_shape=jax.ShapeDtypeStruct((n_atoms, hidden), jnp.float32),
    )(V, mf, woa, wob, b2)

    n_pad = 10240
    n_mols = 256
    hp = jnp.pad(h_atom, ((0, n_pad - n_atoms), (0, 0)))
    bp = jnp.pad(batch, (0, n_pad - n_atoms))
    scatter_pool = _make_scatter_add(n_pad, n_mols, hidden, CH, 4)
    pm = scatter_pool(hp, bp)
    mol_vecs = pl.pallas_call(
        _addp_body,
        grid=(1,),
        in_specs=[pl.BlockSpec((NC, n_mols, hidden), lambda i: (0, 0, 0))],
        out_specs=pl.BlockSpec((n_mols, hidden), lambda i: (0, 0)),
        out_shape=jax.ShapeDtypeStruct((n_mols, hidden), jnp.float32),
    )(pm)

    return (h_atom, batch, mol_vecs, h)


# speedup vs baseline: 2.6941x; 1.0817x over previous
import functools

import jax
import jax.numpy as jnp
from jax import lax
from jax.experimental import pallas as pl
from jax.experimental.pallas import tpu as pltpu
from jax.experimental.pallas import tpu_sc as plsc

NC = 2
NS = 16
NW = NC * NS
LANES = 16

HIDDEN = 128


def _sc_mesh():
    return plsc.VectorSubcoreMesh(core_axis_name="c", subcore_axis_name="s")


def _make_gather(B, D, CH, NBUF):
    b_per_w = B // NW
    n_ch = b_per_w // CH
    n_grp = n_ch // NBUF
    assert b_per_w % CH == 0 and B % (8 * NW) == 0 and CH % 8 == 0
    assert n_ch % NBUF == 0 and CH <= 128

    @functools.partial(
        pl.kernel,
        mesh=_sc_mesh(),
        out_type=jax.ShapeDtypeStruct((B, D), jnp.float32),
        scratch_types=[
            *[pltpu.VMEM((CH,), jnp.int32) for _ in range(NBUF)],
            *[pltpu.VMEM((CH, D), jnp.float32) for _ in range(NBUF)],
            *[pltpu.SemaphoreType.DMA for _ in range(3 * NBUF)],
        ],
    )
    def gather_k(table_hbm, idx_hbm, out_hbm, *rest):
        ibufs = rest[:NBUF]
        bufs = rest[NBUF:2 * NBUF]
        isem = rest[2 * NBUF:3 * NBUF]
        gsem = rest[3 * NBUF:4 * NBUF]
        osem = rest[4 * NBUF:]
        wid = lax.axis_index("s") * NC + lax.axis_index("c")
        base = wid * b_per_w

        def start_idx(c, b):
            pltpu.async_copy(idx_hbm.at[pl.ds(base + c * CH, CH)], ibufs[b],
                             isem[b])

        def wait_idx(b):
            pltpu.make_async_copy(idx_hbm.at[pl.ds(base, CH)], ibufs[b],
                                  isem[b]).wait()

        def start_gather(b):
            pltpu.async_copy(table_hbm.at[ibufs[b]], bufs[b], gsem[b])

        def wait_gather(b):
            pltpu.make_async_copy(table_hbm.at[ibufs[b]], bufs[b],
                                  gsem[b]).wait()

        def start_out(c, b):
            pltpu.async_copy(bufs[b], out_hbm.at[pl.ds(base + c * CH, CH)],
                             osem[b])

        def wait_out(b):
            pltpu.make_async_copy(bufs[b], out_hbm.at[pl.ds(base, CH)],
                                  osem[b]).wait()

        for b in range(NBUF):
            start_idx(b, b)

        def body(g, carry):
            c0 = g * NBUF
            for b in range(NBUF):
                wait_idx(b)

                @pl.when(g > 0)
                def _():
                    wait_out(b)

                start_gather(b)
            for b in range(NBUF):
                wait_gather(b)
                start_out(c0 + b, b)

                @pl.when(g < n_grp - 1)
                def _():
                    start_idx(c0 + NBUF + b, b)

            return carry

        lax.fori_loop(0, n_grp, body, 0)
        for b in range(NBUF):
            wait_out(b)

    return gather_k


def _make_gather_add(B, D, CH, NBUF):
    b_per_w = B // NW
    n_ch = b_per_w // CH
    n_grp = n_ch // NBUF
    assert b_per_w % CH == 0 and B % (8 * NW) == 0 and CH % 8 == 0
    assert n_ch % NBUF == 0 and CH <= 128

    @functools.partial(
        pl.kernel,
        mesh=_sc_mesh(),
        out_type=jax.ShapeDtypeStruct((B, D), jnp.float32),
        scratch_types=[
            *[pltpu.VMEM((CH,), jnp.int32) for _ in range(2 * NBUF)],
            *[pltpu.VMEM((CH, D), jnp.float32) for _ in range(2 * NBUF)],
            *[pltpu.SemaphoreType.DMA for _ in range(3 * NBUF)],
        ],
    )
    def gather_add_k(ta_hbm, ia_hbm, tb_hbm, ib_hbm, out_hbm, *rest):
        iabufs = rest[:NBUF]
        ibbufs = rest[NBUF:2 * NBUF]
        bufs = rest[2 * NBUF:3 * NBUF]
        bufsb = rest[3 * NBUF:4 * NBUF]
        isem = rest[4 * NBUF:5 * NBUF]
        gsem = rest[5 * NBUF:6 * NBUF]
        osem = rest[6 * NBUF:]
        wid = lax.axis_index("s") * NC + lax.axis_index("c")
        base = wid * b_per_w

        def start_idx(c, b):
            pltpu.async_copy(ia_hbm.at[pl.ds(base + c * CH, CH)], iabufs[b],
                             isem[b])
            pltpu.async_copy(ib_hbm.at[pl.ds(base + c * CH, CH)], ibbufs[b],
                             isem[b])

        def wait_idx(b):
            pltpu.make_async_copy(ia_hbm.at[pl.ds(base, CH)], iabufs[b],
                                  isem[b]).wait()
            pltpu.make_async_copy(ib_hbm.at[pl.ds(base, CH)], ibbufs[b],
                                  isem[b]).wait()

        def start_ga(b):
            pltpu.async_copy(ta_hbm.at[iabufs[b]], bufs[b], gsem[b])

        def wait_g(b):
            pltpu.make_async_copy(ta_hbm.at[iabufs[b]], bufs[b],
                                  gsem[b]).wait()

        def start_gb(b):
            pltpu.async_copy(tb_hbm.at[ibbufs[b]], bufsb[b], gsem[b])

        def accum(b):
            def arow(r, carry):
                for j in range(D // LANES):
                    sl = pl.ds(j * LANES, LANES)
                    bufs[b][r, sl] = bufs[b][r, sl] + bufsb[b][r, sl]
                return carry

            lax.fori_loop(0, CH, arow, 0)

        def start_out(c, b):
            pltpu.async_copy(bufs[b], out_hbm.at[pl.ds(base + c * CH, CH)],
                             osem[b])

        def wait_out(b):
            pltpu.make_async_copy(bufs[b], out_hbm.at[pl.ds(base, CH)],
                                  osem[b]).wait()

        for b in range(NBUF):
            start_idx(b, b)

        def body(g, carry):
            c0 = g * NBUF
            for b in range(NBUF):
                wait_idx(b)

                @pl.when(g > 0)
                def _():
                    wait_out(b)

                start_ga(b)
                start_gb(b)
            for b in range(NBUF):
                wait_g(b)
                wait_g(b)
                accum(b)
                start_out(c0 + b, b)

                @pl.when(g < n_grp - 1)
                def _():
                    start_idx(c0 + NBUF + b, b)

            return carry

        lax.fori_loop(0, n_grp, body, 0)
        for b in range(NBUF):
            wait_out(b)

    return gather_add_k


def _make_scatter_add(B, T, D, CH, NBUF):
    b_per_w = B // NW
    n_ch = b_per_w // CH
    n_grp = n_ch // NBUF
    rpt = T // NS
    assert b_per_w % CH == 0 and T % NS == 0 and CH % 8 == 0 and rpt % 8 == 0
    assert n_ch % NBUF == 0 and CH <= 128

    @functools.partial(
        pl.kernel,
        mesh=_sc_mesh(),
        out_type=jax.ShapeDtypeStruct((NC, T, D), jnp.float32),
        scratch_types=[
            *[pltpu.VMEM((CH,), jnp.int32) for _ in range(NBUF)],
            *[pltpu.VMEM((CH, D), jnp.float32) for _ in range(NBUF)],
            pltpu.VMEM_SHARED((T, D), jnp.float32),
            *[pltpu.SemaphoreType.DMA for _ in range(2 * NBUF)],
        ],
    )
    def scatter_k(vals_hbm, dest_hbm, out_hbm, *rest):
        ibufs = rest[:NBUF]
        bufs = rest[NBUF:2 * NBUF]
        acc_sh = rest[2 * NBUF]
        lsem = rest[2 * NBUF + 1:3 * NBUF + 1]
        asem = rest[3 * NBUF + 1:]
        cid = lax.axis_index("c")
        sid = lax.axis_index("s")
        wid = sid * NC + cid
        base = wid * b_per_w

        zero = jnp.zeros((LANES,), jnp.float32)
        zch = CH if rpt % CH == 0 else rpt
        assert rpt % zch == 0 and zch <= CH

        def zrow(r, carry):
            for j in range(D // LANES):
                bufs[0][r, pl.ds(j * LANES, LANES)] = zero
            return carry

        lax.fori_loop(0, zch, zrow, 0)

        def zcopy(k, carry):
            pltpu.sync_copy(
                bufs[0].at[pl.ds(0, zch)],
                acc_sh.at[pl.ds(sid * rpt + k * zch, zch)],
            )
            return carry

        lax.fori_loop(0, rpt // zch, zcopy, 0)
        plsc.subcore_barrier()

        def start_load(c, b):
            pltpu.async_copy(vals_hbm.at[pl.ds(base + c * CH, CH)], bufs[b],
                             lsem[b])
            pltpu.async_copy(dest_hbm.at[pl.ds(base + c * CH, CH)], ibufs[b],
                             lsem[b])

        def wait_load(b):
            pltpu.make_async_copy(vals_hbm.at[pl.ds(base, CH)], bufs[b],
                                  lsem[b]).wait()
            pltpu.make_async_copy(dest_hbm.at[pl.ds(base, CH)], ibufs[b],
                                  lsem[b]).wait()

        def start_add(b):
            pltpu.async_copy(bufs[b], acc_sh.at[ibufs[b]], asem[b], add=True)

        def wait_add(b):
            pltpu.make_async_copy(bufs[b], acc_sh.at[ibufs[b]],
                                  asem[b]).wait()

        for b in range(NBUF):
            start_load(b, b)

        def body(g, carry):
            c0 = g * NBUF
            for b in range(NBUF):
                wait_load(b)
                start_add(b)
            for b in range(NBUF):
                wait_add(b)

                @pl.when(g < n_grp - 1)
                def _():
                    start_load(c0 + NBUF + b, b)

            return carry

        lax.fori_loop(0, n_grp, body, 0)
        plsc.subcore_barrier()
        pltpu.sync_copy(
            acc_sh.at[pl.ds(sid * rpt, rpt)],
            out_hbm.at[cid, pl.ds(sid * rpt, rpt)],
        )

    return scatter_k


def _proj_body(x_ref, w_ref, o_ref):
    o_ref[...] = jnp.dot(x_ref[...], w_ref[...],
                         preferred_element_type=jnp.float32)


def _init_body(asrc_ref, e_ref, wb_ref, w_ref, h0_ref, hw_ref):
    h0 = jnp.maximum(
        asrc_ref[...]
        + jnp.dot(e_ref[...], wb_ref[...], preferred_element_type=jnp.float32),
        0.0,
    )
    h0_ref[...] = h0
    hw_ref[...] = h0 * w_ref[...]


def _addp_body(p_ref, o_ref):
    o_ref[...] = p_ref[0] + p_ref[1]


def _addp_neg_body(p_ref, o_ref):
    o_ref[...] = -(p_ref[0] + p_ref[1])


def _combine_body(d_ref, h0_ref, w_ref, whn_ref, h_ref, hw_ref):
    h = jnp.maximum(
        jnp.dot(d_ref[...], whn_ref[...], preferred_element_type=jnp.float32)
        + h0_ref[...],
        0.0,
    )
    h_ref[...] = h
    hw_ref[...] = h * w_ref[...]


def _atom_body(v_ref, mf_ref, woa_ref, wob_ref, b_ref, o_ref):
    m = mf_ref[0] + mf_ref[1]
    o_ref[...] = jnp.maximum(
        jnp.dot(v_ref[...], woa_ref[...], preferred_element_type=jnp.float32)
        + jnp.dot(m, wob_ref[...], preferred_element_type=jnp.float32)
        + b_ref[...],
        0.0,
    )


def _row_spec(br, d):
    return pl.BlockSpec((br, d), lambda i: (i, 0))


def _full_spec(shape):
    return pl.BlockSpec(shape, lambda i: tuple(0 for _ in shape))


def kernel(V, E, edge_index, rev_edge_index, batch, weight, W_i, W_h, W_o, b_o):
    n_atoms, atom_dim = V.shape
    n_edges, bond_dim = E.shape
    hidden = W_h.shape[0]

    src = edge_index[0]
    dest = edge_index[1]
    w2 = weight[:, None]
    wa = W_i[:atom_dim]
    wb = W_i[atom_dim:]
    woa = W_o[:atom_dim]
    wob = W_o[atom_dim:]
    b2 = b_o[None, :]

    BR_E = 2000
    BR_A = 2000
    CH = 40
    NBUF = 5

    t_acc = 10240
    gather = _make_gather(n_edges, hidden, CH, NBUF)
    scatter_edges = _make_scatter_add(n_edges, t_acc, hidden, CH, NBUF)

    A = pl.pallas_call(
        _proj_body,
        grid=(n_atoms // BR_A,),
        in_specs=[_row_spec(BR_A, atom_dim), _full_spec((atom_dim, hidden))],
        out_specs=_row_spec(BR_A, hidden),
        out_shape=jax.ShapeDtypeStruct((n_atoms, hidden), jnp.float32),
    )(V, wa)

    a_src = gather(A, src)

    h0, hw = pl.pallas_call(
        _init_body,
        grid=(n_edges // BR_E,),
        in_specs=[
            _row_spec(BR_E, hidden),
            _row_spec(BR_E, bond_dim),
            _full_spec((bond_dim, hidden)),
            _row_spec(BR_E, 1),
        ],
        out_specs=[_row_spec(BR_E, hidden), _row_spec(BR_E, hidden)],
        out_shape=[
            jax.ShapeDtypeStruct((n_edges, hidden), jnp.float32),
            jax.ShapeDtypeStruct((n_edges, hidden), jnp.float32),
        ],
    )(a_src, E, wb, w2)

    add_partials_neg = pl.pallas_call(
        _addp_neg_body,
        grid=(t_acc // 2048,),
        in_specs=[pl.BlockSpec((NC, 2048, hidden), lambda i: (0, i, 0))],
        out_specs=_row_spec(2048, hidden),
        out_shape=jax.ShapeDtypeStruct((t_acc, hidden), jnp.float32),
    )

    combine = pl.pallas_call(
        _combine_body,
        grid=(n_edges // BR_E,),
        in_specs=[
            _row_spec(BR_E, hidden),
            _row_spec(BR_E, hidden),
            _row_spec(BR_E, 1),
            _full_spec((hidden, hidden)),
        ],
        out_specs=[_row_spec(BR_E, hidden), _row_spec(BR_E, hidden)],
        out_shape=[
            jax.ShapeDtypeStruct((n_edges, hidden), jnp.float32),
            jax.ShapeDtypeStruct((n_edges, hidden), jnp.float32),
        ],
    )

    gather_add = _make_gather_add(n_edges, hidden, CH, NBUF)
    wh_neg = -W_h

    h = h0
    for _ in range(3):
        s_neg = add_partials_neg(scatter_edges(hw, dest))
        d = gather_add(s_neg, src, hw, rev_edge_index)
        h, hw = combine(d, h0, w2, wh_neg)

    mf = scatter_edges(h, dest)

    h_atom = pl.pallas_call(
        _atom_body,
        grid=(n_atoms // BR_A,),
        in_specs=[
            _row_spec(BR_A, atom_dim),
            pl.BlockSpec((NC, BR_A, hidden), lambda i: (0, i, 0)),
            _full_spec((atom_dim, hidden)),
            _full_spec((hidden, hidden)),
            pl.BlockSpec((1, hidden), lambda i: (0, 0)),
        ],
        out_specs=_row_spec(BR_A, hidden),
        out_shape=jax.ShapeDtypeStruct((n_atoms, hidden), jnp.float32),
    )(V, mf, woa, wob, b2)

    n_pad = 10240
    n_mols = 256
    hp = jnp.pad(h_atom, ((0, n_pad - n_atoms), (0, 0)))
    bp = jnp.pad(batch, (0, n_pad - n_atoms))
    scatter_pool = _make_scatter_add(n_pad, n_mols, hidden, CH, 4)
    pm = scatter_pool(hp, bp)
    mol_vecs = pl.pallas_call(
        _addp_body,
        grid=(1,),
        in_specs=[pl.BlockSpec((NC, n_mols, hidden), lambda i: (0, 0, 0))],
        out_specs=pl.BlockSpec((n_mols, hidden), lambda i: (0, 0)),
        out_shape=jax.ShapeDtypeStruct((n_mols, hidden), jnp.float32),
    )(pm)

    return (h_atom, batch, mol_vecs, h)


# speedup vs baseline: 2.7396x; 1.0169x over previous
import functools

import jax
import jax.numpy as jnp
from jax import lax
from jax.experimental import pallas as pl
from jax.experimental.pallas import tpu as pltpu
from jax.experimental.pallas import tpu_sc as plsc

NC = 2
NS = 16
NW = NC * NS
LANES = 16

HIDDEN = 128


def _sc_mesh():
    return plsc.VectorSubcoreMesh(core_axis_name="c", subcore_axis_name="s")


def _make_gather(B, D, CH, NBUF):
    b_per_w = B // NW
    n_ch = b_per_w // CH
    n_grp = n_ch // NBUF
    assert b_per_w % CH == 0 and B % (8 * NW) == 0 and CH % 8 == 0
    assert n_ch % NBUF == 0 and CH <= 128

    @functools.partial(
        pl.kernel,
        mesh=_sc_mesh(),
        out_type=jax.ShapeDtypeStruct((B, D), jnp.float32),
        scratch_types=[
            *[pltpu.VMEM((CH,), jnp.int32) for _ in range(NBUF)],
            *[pltpu.VMEM((CH, D), jnp.float32) for _ in range(NBUF)],
            *[pltpu.SemaphoreType.DMA for _ in range(3 * NBUF)],
        ],
    )
    def gather_k(table_hbm, idx_hbm, out_hbm, *rest):
        ibufs = rest[:NBUF]
        bufs = rest[NBUF:2 * NBUF]
        isem = rest[2 * NBUF:3 * NBUF]
        gsem = rest[3 * NBUF:4 * NBUF]
        osem = rest[4 * NBUF:]
        wid = lax.axis_index("s") * NC + lax.axis_index("c")
        base = wid * b_per_w

        def start_idx(c, b):
            pltpu.async_copy(idx_hbm.at[pl.ds(base + c * CH, CH)], ibufs[b],
                             isem[b])

        def wait_idx(b):
            pltpu.make_async_copy(idx_hbm.at[pl.ds(base, CH)], ibufs[b],
                                  isem[b]).wait()

        def start_gather(b):
            pltpu.async_copy(table_hbm.at[ibufs[b]], bufs[b], gsem[b])

        def wait_gather(b):
            pltpu.make_async_copy(table_hbm.at[ibufs[b]], bufs[b],
                                  gsem[b]).wait()

        def start_out(c, b):
            pltpu.async_copy(bufs[b], out_hbm.at[pl.ds(base + c * CH, CH)],
                             osem[b])

        def wait_out(b):
            pltpu.make_async_copy(bufs[b], out_hbm.at[pl.ds(base, CH)],
                                  osem[b]).wait()

        for b in range(NBUF):
            start_idx(b, b)

        def body(g, carry):
            c0 = g * NBUF
            for b in range(NBUF):
                wait_idx(b)

                @pl.when(g > 0)
                def _():
                    wait_out(b)

                start_gather(b)
            for b in range(NBUF):
                wait_gather(b)
                start_out(c0 + b, b)

                @pl.when(g < n_grp - 1)
                def _():
                    start_idx(c0 + NBUF + b, b)

            return carry

        lax.fori_loop(0, n_grp, body, 0)
        for b in range(NBUF):
            wait_out(b)

    return gather_k


def _make_gather_add(B, D, CH, NBUF):
    b_per_w = B // NW
    n_ch = b_per_w // CH
    n_grp = n_ch // NBUF
    assert b_per_w % CH == 0 and B % (8 * NW) == 0 and CH % 8 == 0
    assert n_ch % NBUF == 0 and CH <= 128

    @functools.partial(
        pl.kernel,
        mesh=_sc_mesh(),
        out_type=jax.ShapeDtypeStruct((B, D), jnp.float32),
        scratch_types=[
            *[pltpu.VMEM((CH,), jnp.int32) for _ in range(2 * NBUF)],
            *[pltpu.VMEM((CH, D), jnp.float32) for _ in range(2 * NBUF)],
            *[pltpu.SemaphoreType.DMA for _ in range(3 * NBUF)],
        ],
    )
    def gather_add_k(ta_hbm, ia_hbm, tb_hbm, ib_hbm, out_hbm, *rest):
        iabufs = rest[:NBUF]
        ibbufs = rest[NBUF:2 * NBUF]
        bufs = rest[2 * NBUF:3 * NBUF]
        bufsb = rest[3 * NBUF:4 * NBUF]
        isem = rest[4 * NBUF:5 * NBUF]
        gsem = rest[5 * NBUF:6 * NBUF]
        osem = rest[6 * NBUF:]
        wid = lax.axis_index("s") * NC + lax.axis_index("c")
        base = wid * b_per_w

        def start_idx(c, b):
            pltpu.async_copy(ia_hbm.at[pl.ds(base + c * CH, CH)], iabufs[b],
                             isem[b])
            pltpu.async_copy(ib_hbm.at[pl.ds(base + c * CH, CH)], ibbufs[b],
                             isem[b])

        def wait_idx(b):
            pltpu.make_async_copy(ia_hbm.at[pl.ds(base, CH)], iabufs[b],
                                  isem[b]).wait()
            pltpu.make_async_copy(ib_hbm.at[pl.ds(base, CH)], ibbufs[b],
                                  isem[b]).wait()

        def start_ga(b):
            pltpu.async_copy(ta_hbm.at[iabufs[b]], bufs[b], gsem[b])

        def wait_g(b):
            pltpu.make_async_copy(ta_hbm.at[iabufs[b]], bufs[b],
                                  gsem[b]).wait()

        def start_gb(b):
            pltpu.async_copy(tb_hbm.at[ibbufs[b]], bufsb[b], gsem[b])

        def accum(b):
            def arow(r, carry):
                for j in range(D // LANES):
                    sl = pl.ds(j * LANES, LANES)
                    bufs[b][r, sl] = bufs[b][r, sl] + bufsb[b][r, sl]
                return carry

            lax.fori_loop(0, CH, arow, 0)

        def start_out(c, b):
            pltpu.async_copy(bufs[b], out_hbm.at[pl.ds(base + c * CH, CH)],
                             osem[b])

        def wait_out(b):
            pltpu.make_async_copy(bufs[b], out_hbm.at[pl.ds(base, CH)],
                                  osem[b]).wait()

        for b in range(NBUF):
            start_idx(b, b)

        def body(g, carry):
            c0 = g * NBUF
            for b in range(NBUF):
                wait_idx(b)

                @pl.when(g > 0)
                def _():
                    wait_out(b)

                start_ga(b)
                start_gb(b)
            for b in range(NBUF):
                wait_g(b)
                wait_g(b)
                accum(b)
                start_out(c0 + b, b)

                @pl.when(g < n_grp - 1)
                def _():
                    start_idx(c0 + NBUF + b, b)

            return carry

        lax.fori_loop(0, n_grp, body, 0)
        for b in range(NBUF):
            wait_out(b)

    return gather_add_k


def _make_scatter_add(B, T, D, CH, NBUF):
    b_per_w = B // NW
    n_ch = b_per_w // CH
    n_grp = n_ch // NBUF
    rpt = T // NS
    assert b_per_w % CH == 0 and T % NS == 0 and CH % 8 == 0 and rpt % 8 == 0
    assert n_ch % NBUF == 0 and CH <= 128

    @functools.partial(
        pl.kernel,
        mesh=_sc_mesh(),
        out_type=jax.ShapeDtypeStruct((NC, T, D), jnp.float32),
        scratch_types=[
            *[pltpu.VMEM((CH,), jnp.int32) for _ in range(NBUF)],
            *[pltpu.VMEM((CH, D), jnp.float32) for _ in range(NBUF)],
            pltpu.VMEM_SHARED((T, D), jnp.float32),
            *[pltpu.SemaphoreType.DMA for _ in range(2 * NBUF)],
        ],
    )
    def scatter_k(vals_hbm, dest_hbm, out_hbm, *rest):
        ibufs = rest[:NBUF]
        bufs = rest[NBUF:2 * NBUF]
        acc_sh = rest[2 * NBUF]
        lsem = rest[2 * NBUF + 1:3 * NBUF + 1]
        asem = rest[3 * NBUF + 1:]
        cid = lax.axis_index("c")
        sid = lax.axis_index("s")
        wid = sid * NC + cid
        base = wid * b_per_w

        zero = jnp.zeros((LANES,), jnp.float32)
        zch = CH if rpt % CH == 0 else rpt
        assert rpt % zch == 0 and zch <= CH

        def zrow(r, carry):
            for j in range(D // LANES):
                bufs[0][r, pl.ds(j * LANES, LANES)] = zero
            return carry

        def zcopy(k, carry):
            pltpu.sync_copy(
                bufs[0].at[pl.ds(0, zch)],
                acc_sh.at[pl.ds(sid * rpt + k * zch, zch)],
            )
            return carry

        def start_load(c, b):
            pltpu.async_copy(vals_hbm.at[pl.ds(base + c * CH, CH)], bufs[b],
                             lsem[b])
            pltpu.async_copy(dest_hbm.at[pl.ds(base + c * CH, CH)], ibufs[b],
                             lsem[b])

        def wait_load(b):
            pltpu.make_async_copy(vals_hbm.at[pl.ds(base, CH)], bufs[b],
                                  lsem[b]).wait()
            pltpu.make_async_copy(dest_hbm.at[pl.ds(base, CH)], ibufs[b],
                                  lsem[b]).wait()

        def start_add(b):
            pltpu.async_copy(bufs[b], acc_sh.at[ibufs[b]], asem[b], add=True)

        def wait_add(b):
            pltpu.make_async_copy(bufs[b], acc_sh.at[ibufs[b]],
                                  asem[b]).wait()

        for b in range(1, NBUF):
            start_load(b, b)
        lax.fori_loop(0, zch, zrow, 0)
        lax.fori_loop(0, rpt // zch, zcopy, 0)
        start_load(0, 0)
        plsc.subcore_barrier()

        def body(g, carry):
            c0 = g * NBUF
            for b in range(NBUF):
                wait_load(b)
                start_add(b)
            for b in range(NBUF):
                wait_add(b)

                @pl.when(g < n_grp - 1)
                def _():
                    start_load(c0 + NBUF + b, b)

            return carry

        lax.fori_loop(0, n_grp, body, 0)
        plsc.subcore_barrier()
        pltpu.sync_copy(
            acc_sh.at[pl.ds(sid * rpt, rpt)],
            out_hbm.at[cid, pl.ds(sid * rpt, rpt)],
        )

    return scatter_k


def _proj_body(x_ref, w_ref, o_ref):
    o_ref[...] = jnp.dot(x_ref[...], w_ref[...],
                         preferred_element_type=jnp.float32)


def _init_body(asrc_ref, e_ref, wb_ref, w_ref, h0_ref, hw_ref):
    h0 = jnp.maximum(
        asrc_ref[...]
        + jnp.dot(e_ref[...], wb_ref[...], preferred_element_type=jnp.float32),
        0.0,
    )
    h0_ref[...] = h0
    hw_ref[...] = h0 * w_ref[...]


def _addp_body(p_ref, o_ref):
    o_ref[...] = p_ref[0] + p_ref[1]


def _addp_neg_body(p_ref, o_ref):
    o_ref[...] = -(p_ref[0] + p_ref[1])


def _combine_body(d_ref, h0_ref, w_ref, whn_ref, h_ref, hw_ref):
    h = jnp.maximum(
        jnp.dot(d_ref[...], whn_ref[...], preferred_element_type=jnp.float32)
        + h0_ref[...],
        0.0,
    )
    h_ref[...] = h
    hw_ref[...] = h * w_ref[...]


def _atom_body(v_ref, mf_ref, woa_ref, wob_ref, b_ref, o_ref):
    m = mf_ref[0] + mf_ref[1]
    o_ref[...] = jnp.maximum(
        jnp.dot(v_ref[...], woa_ref[...], preferred_element_type=jnp.float32)
        + jnp.dot(m, wob_ref[...], preferred_element_type=jnp.float32)
        + b_ref[...],
        0.0,
    )


def _row_spec(br, d):
    return pl.BlockSpec((br, d), lambda i: (i, 0))


def _full_spec(shape):
    return pl.BlockSpec(shape, lambda i: tuple(0 for _ in shape))


def kernel(V, E, edge_index, rev_edge_index, batch, weight, W_i, W_h, W_o, b_o):
    n_atoms, atom_dim = V.shape
    n_edges, bond_dim = E.shape
    hidden = W_h.shape[0]

    src = edge_index[0]
    dest = edge_index[1]
    w2 = weight[:, None]
    wa = W_i[:atom_dim]
    wb = W_i[atom_dim:]
    woa = W_o[:atom_dim]
    wob = W_o[atom_dim:]
    b2 = b_o[None, :]

    BR_E = 2000
    BR_A = 2000
    CH = 80
    CHS = 40
    NBUF = 5

    t_acc = 10240
    gather = _make_gather(n_edges, hidden, CH, NBUF)
    scatter_edges = _make_scatter_add(n_edges, t_acc, hidden, CHS, NBUF)

    A = pl.pallas_call(
        _proj_body,
        grid=(n_atoms // BR_A,),
        in_specs=[_row_spec(BR_A, atom_dim), _full_spec((atom_dim, hidden))],
        out_specs=_row_spec(BR_A, hidden),
        out_shape=jax.ShapeDtypeStruct((n_atoms, hidden), jnp.float32),
    )(V, wa)

    a_src = gather(A, src)

    h0, hw = pl.pallas_call(
        _init_body,
        grid=(n_edges // BR_E,),
        in_specs=[
            _row_spec(BR_E, hidden),
            _row_spec(BR_E, bond_dim),
            _full_spec((bond_dim, hidden)),
            _row_spec(BR_E, 1),
        ],
        out_specs=[_row_spec(BR_E, hidden), _row_spec(BR_E, hidden)],
        out_shape=[
            jax.ShapeDtypeStruct((n_edges, hidden), jnp.float32),
            jax.ShapeDtypeStruct((n_edges, hidden), jnp.float32),
        ],
    )(a_src, E, wb, w2)

    add_partials_neg = pl.pallas_call(
        _addp_neg_body,
        grid=(t_acc // 2048,),
        in_specs=[pl.BlockSpec((NC, 2048, hidden), lambda i: (0, i, 0))],
        out_specs=_row_spec(2048, hidden),
        out_shape=jax.ShapeDtypeStruct((t_acc, hidden), jnp.float32),
    )

    combine = pl.pallas_call(
        _combine_body,
        grid=(n_edges // BR_E,),
        in_specs=[
            _row_spec(BR_E, hidden),
            _row_spec(BR_E, hidden),
            _row_spec(BR_E, 1),
            _full_spec((hidden, hidden)),
        ],
        out_specs=[_row_spec(BR_E, hidden), _row_spec(BR_E, hidden)],
        out_shape=[
            jax.ShapeDtypeStruct((n_edges, hidden), jnp.float32),
            jax.ShapeDtypeStruct((n_edges, hidden), jnp.float32),
        ],
    )

    gather_add = _make_gather_add(n_edges, hidden, CH, NBUF)
    wh_neg = -W_h

    h = h0
    for _ in range(3):
        s_neg = add_partials_neg(scatter_edges(hw, dest))
        d = gather_add(s_neg, src, hw, rev_edge_index)
        h, hw = combine(d, h0, w2, wh_neg)

    mf = scatter_edges(h, dest)

    h_atom = pl.pallas_call(
        _atom_body,
        grid=(n_atoms // BR_A,),
        in_specs=[
            _row_spec(BR_A, atom_dim),
            pl.BlockSpec((NC, BR_A, hidden), lambda i: (0, i, 0)),
            _full_spec((atom_dim, hidden)),
            _full_spec((hidden, hidden)),
            pl.BlockSpec((1, hidden), lambda i: (0, 0)),
        ],
        out_specs=_row_spec(BR_A, hidden),
        out_shape=jax.ShapeDtypeStruct((n_atoms, hidden), jnp.float32),
    )(V, mf, woa, wob, b2)

    n_pad = 10240
    n_mols = 256
    hp = jnp.pad(h_atom, ((0, n_pad - n_atoms), (0, 0)))
    bp = jnp.pad(batch, (0, n_pad - n_atoms))
    scatter_pool = _make_scatter_add(n_pad, n_mols, hidden, CHS, 4)
    pm = scatter_pool(hp, bp)
    mol_vecs = pl.pallas_call(
        _addp_body,
        grid=(1,),
        in_specs=[pl.BlockSpec((NC, n_mols, hidden), lambda i: (0, 0, 0))],
        out_specs=pl.BlockSpec((n_mols, hidden), lambda i: (0, 0)),
        out_shape=jax.ShapeDtypeStruct((n_mols, hidden), jnp.float32),
    )(pm)

    return (h_atom, batch, mol_vecs, h)


# speedup vs baseline: 2.7492x; 1.0035x over previous
import functools

import jax
import jax.numpy as jnp
from jax import lax
from jax.experimental import pallas as pl
from jax.experimental.pallas import tpu as pltpu
from jax.experimental.pallas import tpu_sc as plsc

NC = 2
NS = 16
NW = NC * NS
LANES = 16

HIDDEN = 128


def _sc_mesh():
    return plsc.VectorSubcoreMesh(core_axis_name="c", subcore_axis_name="s")


def _make_gather(B, D, CH, NBUF):
    b_per_w = B // NW
    n_ch = b_per_w // CH
    n_grp = n_ch // NBUF
    assert b_per_w % CH == 0 and B % (8 * NW) == 0 and CH % 8 == 0
    assert n_ch % NBUF == 0 and CH <= 128

    @functools.partial(
        pl.kernel,
        mesh=_sc_mesh(),
        out_type=jax.ShapeDtypeStruct((B, D), jnp.float32),
        scratch_types=[
            *[pltpu.VMEM((CH,), jnp.int32) for _ in range(NBUF)],
            *[pltpu.VMEM((CH, D), jnp.float32) for _ in range(NBUF)],
            *[pltpu.SemaphoreType.DMA for _ in range(3 * NBUF)],
        ],
    )
    def gather_k(table_hbm, idx_hbm, out_hbm, *rest):
        ibufs = rest[:NBUF]
        bufs = rest[NBUF:2 * NBUF]
        isem = rest[2 * NBUF:3 * NBUF]
        gsem = rest[3 * NBUF:4 * NBUF]
        osem = rest[4 * NBUF:]
        wid = lax.axis_index("s") * NC + lax.axis_index("c")
        base = wid * b_per_w

        def start_idx(c, b):
            pltpu.async_copy(idx_hbm.at[pl.ds(base + c * CH, CH)], ibufs[b],
                             isem[b])

        def wait_idx(b):
            pltpu.make_async_copy(idx_hbm.at[pl.ds(base, CH)], ibufs[b],
                                  isem[b]).wait()

        def start_gather(b):
            pltpu.async_copy(table_hbm.at[ibufs[b]], bufs[b], gsem[b])

        def wait_gather(b):
            pltpu.make_async_copy(table_hbm.at[ibufs[b]], bufs[b],
                                  gsem[b]).wait()

        def start_out(c, b):
            pltpu.async_copy(bufs[b], out_hbm.at[pl.ds(base + c * CH, CH)],
                             osem[b])

        def wait_out(b):
            pltpu.make_async_copy(bufs[b], out_hbm.at[pl.ds(base, CH)],
                                  osem[b]).wait()

        for b in range(NBUF):
            start_idx(b, b)

        def body(g, carry):
            c0 = g * NBUF
            for b in range(NBUF):
                wait_idx(b)

                @pl.when(g > 0)
                def _():
                    wait_out(b)

                start_gather(b)
            for b in range(NBUF):
                wait_gather(b)
                start_out(c0 + b, b)

                @pl.when(g < n_grp - 1)
                def _():
                    start_idx(c0 + NBUF + b, b)

            return carry

        lax.fori_loop(0, n_grp, body, 0)
        for b in range(NBUF):
            wait_out(b)

    return gather_k


def _make_gather_add(B, D, CH, NBUF):
    b_per_w = B // NW
    n_ch = b_per_w // CH
    n_grp = n_ch // NBUF
    assert b_per_w % CH == 0 and B % (8 * NW) == 0 and CH % 8 == 0
    assert n_ch % NBUF == 0 and CH <= 128

    @functools.partial(
        pl.kernel,
        mesh=_sc_mesh(),
        out_type=jax.ShapeDtypeStruct((B, D), jnp.float32),
        scratch_types=[
            *[pltpu.VMEM((CH,), jnp.int32) for _ in range(2 * NBUF)],
            *[pltpu.VMEM((CH, D), jnp.float32) for _ in range(2 * NBUF)],
            *[pltpu.SemaphoreType.DMA for _ in range(3 * NBUF)],
        ],
    )
    def gather_add_k(ta_hbm, ia_hbm, tb_hbm, ib_hbm, out_hbm, *rest):
        iabufs = rest[:NBUF]
        ibbufs = rest[NBUF:2 * NBUF]
        bufs = rest[2 * NBUF:3 * NBUF]
        bufsb = rest[3 * NBUF:4 * NBUF]
        isem = rest[4 * NBUF:5 * NBUF]
        gsem = rest[5 * NBUF:6 * NBUF]
        osem = rest[6 * NBUF:]
        wid = lax.axis_index("s") * NC + lax.axis_index("c")
        base = wid * b_per_w

        def start_idx(c, b):
            pltpu.async_copy(ia_hbm.at[pl.ds(base + c * CH, CH)], iabufs[b],
                             isem[b])
            pltpu.async_copy(ib_hbm.at[pl.ds(base + c * CH, CH)], ibbufs[b],
                             isem[b])

        def wait_idx(b):
            pltpu.make_async_copy(ia_hbm.at[pl.ds(base, CH)], iabufs[b],
                                  isem[b]).wait()
            pltpu.make_async_copy(ib_hbm.at[pl.ds(base, CH)], ibbufs[b],
                                  isem[b]).wait()

        def start_ga(b):
            pltpu.async_copy(ta_hbm.at[iabufs[b]], bufs[b], gsem[b])

        def wait_g(b):
            pltpu.make_async_copy(ta_hbm.at[iabufs[b]], bufs[b],
                                  gsem[b]).wait()

        def start_gb(b):
            pltpu.async_copy(tb_hbm.at[ibbufs[b]], bufsb[b], gsem[b])

        def accum(b):
            def arow(r, carry):
                for j in range(D // LANES):
                    sl = pl.ds(j * LANES, LANES)
                    bufs[b][r, sl] = bufs[b][r, sl] + bufsb[b][r, sl]
                return carry

            lax.fori_loop(0, CH, arow, 0)

        def start_out(c, b):
            pltpu.async_copy(bufs[b], out_hbm.at[pl.ds(base + c * CH, CH)],
                             osem[b])

        def wait_out(b):
            pltpu.make_async_copy(bufs[b], out_hbm.at[pl.ds(base, CH)],
                                  osem[b]).wait()

        for b in range(NBUF):
            start_idx(b, b)

        def body(g, carry):
            c0 = g * NBUF
            for b in range(NBUF):
                wait_idx(b)

                @pl.when(g > 0)
                def _():
                    wait_out(b)

                start_ga(b)
                start_gb(b)
            for b in range(NBUF):
                wait_g(b)
                wait_g(b)
                accum(b)
                start_out(c0 + b, b)

                @pl.when(g < n_grp - 1)
                def _():
                    start_idx(c0 + NBUF + b, b)

            return carry

        lax.fori_loop(0, n_grp, body, 0)
        for b in range(NBUF):
            wait_out(b)

    return gather_add_k


def _make_scatter_add(B, T, D, CH, NBUF):
    b_per_w = B // NW
    n_ch = b_per_w // CH
    n_grp = n_ch // NBUF
    rpt = T // NS
    assert b_per_w % CH == 0 and T % NS == 0 and CH % 8 == 0 and rpt % 8 == 0
    assert n_ch % NBUF == 0 and CH <= 128

    @functools.partial(
        pl.kernel,
        mesh=_sc_mesh(),
        out_type=jax.ShapeDtypeStruct((NC, T, D), jnp.float32),
        scratch_types=[
            *[pltpu.VMEM((CH,), jnp.int32) for _ in range(NBUF)],
            *[pltpu.VMEM((CH, D), jnp.float32) for _ in range(NBUF)],
            pltpu.VMEM_SHARED((T, D), jnp.float32),
            *[pltpu.SemaphoreType.DMA for _ in range(2 * NBUF)],
        ],
    )
    def scatter_k(vals_hbm, dest_hbm, out_hbm, *rest):
        ibufs = rest[:NBUF]
        bufs = rest[NBUF:2 * NBUF]
        acc_sh = rest[2 * NBUF]
        lsem = rest[2 * NBUF + 1:3 * NBUF + 1]
        asem = rest[3 * NBUF + 1:]
        cid = lax.axis_index("c")
        sid = lax.axis_index("s")
        wid = sid * NC + cid
        base = wid * b_per_w

        zero = jnp.zeros((LANES,), jnp.float32)
        zch = CH if rpt % CH == 0 else rpt
        assert rpt % zch == 0 and zch <= CH

        def zrow(r, carry):
            for j in range(D // LANES):
                bufs[0][r, pl.ds(j * LANES, LANES)] = zero
            return carry

        def zcopy(k, carry):
            pltpu.sync_copy(
                bufs[0].at[pl.ds(0, zch)],
                acc_sh.at[pl.ds(sid * rpt + k * zch, zch)],
            )
            return carry

        def start_load(c, b):
            pltpu.async_copy(vals_hbm.at[pl.ds(base + c * CH, CH)], bufs[b],
                             lsem[b])
            pltpu.async_copy(dest_hbm.at[pl.ds(base + c * CH, CH)], ibufs[b],
                             lsem[b])

        def wait_load(b):
            pltpu.make_async_copy(vals_hbm.at[pl.ds(base, CH)], bufs[b],
                                  lsem[b]).wait()
            pltpu.make_async_copy(dest_hbm.at[pl.ds(base, CH)], ibufs[b],
                                  lsem[b]).wait()

        def start_add(b):
            pltpu.async_copy(bufs[b], acc_sh.at[ibufs[b]], asem[b], add=True)

        def wait_add(b):
            pltpu.make_async_copy(bufs[b], acc_sh.at[ibufs[b]],
                                  asem[b]).wait()

        for b in range(1, NBUF):
            start_load(b, b)
        lax.fori_loop(0, zch, zrow, 0)
        lax.fori_loop(0, rpt // zch, zcopy, 0)
        start_load(0, 0)
        plsc.subcore_barrier()

        def body(g, carry):
            c0 = g * NBUF
            for b in range(NBUF):
                wait_load(b)
                start_add(b)
            for b in range(NBUF):
                wait_add(b)

                @pl.when(g < n_grp - 1)
                def _():
                    start_load(c0 + NBUF + b, b)

            return carry

        lax.fori_loop(0, n_grp, body, 0)
        plsc.subcore_barrier()
        pltpu.sync_copy(
            acc_sh.at[pl.ds(sid * rpt, rpt)],
            out_hbm.at[cid, pl.ds(sid * rpt, rpt)],
        )

    return scatter_k


def _proj_body(x_ref, w_ref, o_ref):
    o_ref[...] = jnp.dot(x_ref[...], w_ref[...],
                         preferred_element_type=jnp.float32)


def _init_body(asrc_ref, e_ref, wb_ref, w_ref, h0_ref, hw_ref):
    h0 = jnp.maximum(
        asrc_ref[...]
        + jnp.dot(e_ref[...], wb_ref[...], preferred_element_type=jnp.float32),
        0.0,
    )
    h0_ref[...] = h0
    hw_ref[...] = h0 * w_ref[...]


def _addp_body(p_ref, o_ref):
    o_ref[...] = p_ref[0] + p_ref[1]


def _addp_neg_body(p_ref, o_ref):
    o_ref[...] = -(p_ref[0] + p_ref[1])


def _combine_body(d_ref, h0_ref, w_ref, whn_ref, h_ref, hw_ref):
    h = jnp.maximum(
        jnp.dot(d_ref[...], whn_ref[...], preferred_element_type=jnp.float32)
        + h0_ref[...],
        0.0,
    )
    h_ref[...] = h
    hw_ref[...] = h * w_ref[...]


def _pool_body(n_mols):
    def body(b_ref, h_ref, o_ref):
        i = pl.program_id(0)
        br = b_ref.shape[2]
        mol_ids = lax.broadcasted_iota(jnp.int32, (n_mols, br), 0)
        onehot = (mol_ids == b_ref[0]).astype(jnp.float32)
        r = jnp.dot(onehot, h_ref[...], preferred_element_type=jnp.float32)

        @pl.when(i == 0)
        def _():
            o_ref[...] = r

        @pl.when(i > 0)
        def _():
            o_ref[...] += r

    return body


def _atom_body(v_ref, mf_ref, woa_ref, wob_ref, b_ref, o_ref):
    m = mf_ref[0] + mf_ref[1]
    o_ref[...] = jnp.maximum(
        jnp.dot(v_ref[...], woa_ref[...], preferred_element_type=jnp.float32)
        + jnp.dot(m, wob_ref[...], preferred_element_type=jnp.float32)
        + b_ref[...],
        0.0,
    )


def _row_spec(br, d):
    return pl.BlockSpec((br, d), lambda i: (i, 0))


def _full_spec(shape):
    return pl.BlockSpec(shape, lambda i: tuple(0 for _ in shape))


def kernel(V, E, edge_index, rev_edge_index, batch, weight, W_i, W_h, W_o, b_o):
    n_atoms, atom_dim = V.shape
    n_edges, bond_dim = E.shape
    hidden = W_h.shape[0]

    src = edge_index[0]
    dest = edge_index[1]
    w2 = weight[:, None]
    wa = W_i[:atom_dim]
    wb = W_i[atom_dim:]
    woa = W_o[:atom_dim]
    wob = W_o[atom_dim:]
    b2 = b_o[None, :]

    BR_E = 2000
    BR_A = 2000
    CH = 80
    CHS = 40
    NBUF = 5

    t_acc = 10240
    gather = _make_gather(n_edges, hidden, CH, NBUF)
    scatter_edges = _make_scatter_add(n_edges, t_acc, hidden, CHS, NBUF)

    A = pl.pallas_call(
        _proj_body,
        grid=(n_atoms // BR_A,),
        in_specs=[_row_spec(BR_A, atom_dim), _full_spec((atom_dim, hidden))],
        out_specs=_row_spec(BR_A, hidden),
        out_shape=jax.ShapeDtypeStruct((n_atoms, hidden), jnp.float32),
    )(V, wa)

    a_src = gather(A, src)

    h0, hw = pl.pallas_call(
        _init_body,
        grid=(n_edges // BR_E,),
        in_specs=[
            _row_spec(BR_E, hidden),
            _row_spec(BR_E, bond_dim),
            _full_spec((bond_dim, hidden)),
            _row_spec(BR_E, 1),
        ],
        out_specs=[_row_spec(BR_E, hidden), _row_spec(BR_E, hidden)],
        out_shape=[
            jax.ShapeDtypeStruct((n_edges, hidden), jnp.float32),
            jax.ShapeDtypeStruct((n_edges, hidden), jnp.float32),
        ],
    )(a_src, E, wb, w2)

    add_partials_neg = pl.pallas_call(
        _addp_neg_body,
        grid=(t_acc // 2048,),
        in_specs=[pl.BlockSpec((NC, 2048, hidden), lambda i: (0, i, 0))],
        out_specs=_row_spec(2048, hidden),
        out_shape=jax.ShapeDtypeStruct((t_acc, hidden), jnp.float32),
    )

    combine = pl.pallas_call(
        _combine_body,
        grid=(n_edges // BR_E,),
        in_specs=[
            _row_spec(BR_E, hidden),
            _row_spec(BR_E, hidden),
            _row_spec(BR_E, 1),
            _full_spec((hidden, hidden)),
        ],
        out_specs=[_row_spec(BR_E, hidden), _row_spec(BR_E, hidden)],
        out_shape=[
            jax.ShapeDtypeStruct((n_edges, hidden), jnp.float32),
            jax.ShapeDtypeStruct((n_edges, hidden), jnp.float32),
        ],
    )

    gather_add = _make_gather_add(n_edges, hidden, CH, NBUF)
    wh_neg = -W_h

    h = h0
    for _ in range(3):
        s_neg = add_partials_neg(scatter_edges(hw, dest))
        d = gather_add(s_neg, src, hw, rev_edge_index)
        h, hw = combine(d, h0, w2, wh_neg)

    mf = scatter_edges(h, dest)

    h_atom = pl.pallas_call(
        _atom_body,
        grid=(n_atoms // BR_A,),
        in_specs=[
            _row_spec(BR_A, atom_dim),
            pl.BlockSpec((NC, BR_A, hidden), lambda i: (0, i, 0)),
            _full_spec((atom_dim, hidden)),
            _full_spec((hidden, hidden)),
            pl.BlockSpec((1, hidden), lambda i: (0, 0)),
        ],
        out_specs=_row_spec(BR_A, hidden),
        out_shape=jax.ShapeDtypeStruct((n_atoms, hidden), jnp.float32),
    )(V, mf, woa, wob, b2)

    n_mols = 256
    b2d = batch.reshape(n_atoms // BR_A, 1, BR_A)
    mol_vecs = pl.pallas_call(
        _pool_body(n_mols),
        grid=(n_atoms // BR_A,),
        in_specs=[
            pl.BlockSpec((1, 1, BR_A), lambda i: (i, 0, 0)),
            _row_spec(BR_A, hidden),
        ],
        out_specs=pl.BlockSpec((n_mols, hidden), lambda i: (0, 0)),
        out_shape=jax.ShapeDtypeStruct((n_mols, hidden), jnp.float32),
    )(b2d, h_atom)

    return (h_atom, batch, mol_vecs, h)


# speedup vs baseline: 2.8312x; 1.0298x over previous
import functools

import jax
import jax.numpy as jnp
from jax import lax
from jax.experimental import pallas as pl
from jax.experimental.pallas import tpu as pltpu
from jax.experimental.pallas import tpu_sc as plsc

NC = 2
NS = 16
NW = NC * NS
LANES = 16

HIDDEN = 128


def _sc_mesh():
    return plsc.VectorSubcoreMesh(core_axis_name="c", subcore_axis_name="s")


def _make_gather(B, D, CH, NBUF):
    b_per_w = B // NW
    n_ch = b_per_w // CH
    n_grp = n_ch // NBUF
    assert b_per_w % CH == 0 and B % (8 * NW) == 0 and CH % 8 == 0
    assert n_ch % NBUF == 0 and CH <= 128

    @functools.partial(
        pl.kernel,
        mesh=_sc_mesh(),
        out_type=jax.ShapeDtypeStruct((B, D), jnp.float32),
        scratch_types=[
            *[pltpu.VMEM((CH,), jnp.int32) for _ in range(NBUF)],
            *[pltpu.VMEM((CH, D), jnp.float32) for _ in range(NBUF)],
            *[pltpu.SemaphoreType.DMA for _ in range(3 * NBUF)],
        ],
    )
    def gather_k(table_hbm, idx_hbm, out_hbm, *rest):
        ibufs = rest[:NBUF]
        bufs = rest[NBUF:2 * NBUF]
        isem = rest[2 * NBUF:3 * NBUF]
        gsem = rest[3 * NBUF:4 * NBUF]
        osem = rest[4 * NBUF:]
        wid = lax.axis_index("s") * NC + lax.axis_index("c")
        base = wid * b_per_w

        def start_idx(c, b):
            pltpu.async_copy(idx_hbm.at[pl.ds(base + c * CH, CH)], ibufs[b],
                             isem[b])

        def wait_idx(b):
            pltpu.make_async_copy(idx_hbm.at[pl.ds(base, CH)], ibufs[b],
                                  isem[b]).wait()

        def start_gather(b):
            pltpu.async_copy(table_hbm.at[ibufs[b]], bufs[b], gsem[b])

        def wait_gather(b):
            pltpu.make_async_copy(table_hbm.at[ibufs[b]], bufs[b],
                                  gsem[b]).wait()

        def start_out(c, b):
            pltpu.async_copy(bufs[b], out_hbm.at[pl.ds(base + c * CH, CH)],
                             osem[b])

        def wait_out(b):
            pltpu.make_async_copy(bufs[b], out_hbm.at[pl.ds(base, CH)],
                                  osem[b]).wait()

        for b in range(NBUF):
            start_idx(b, b)

        def body(g, carry):
            c0 = g * NBUF
            for b in range(NBUF):
                wait_idx(b)

                @pl.when(g > 0)
                def _():
                    wait_out(b)

                start_gather(b)
            for b in range(NBUF):
                wait_gather(b)
                start_out(c0 + b, b)

                @pl.when(g < n_grp - 1)
                def _():
                    start_idx(c0 + NBUF + b, b)

            return carry

        lax.fori_loop(0, n_grp, body, 0)
        for b in range(NBUF):
            wait_out(b)

    return gather_k


def _make_gather_add(B, T, D, CH, NBUF):
    b_per_w = B // NW
    n_ch = b_per_w // CH
    n_grp = n_ch // NBUF
    rpt = T // NS
    assert b_per_w % CH == 0 and B % (8 * NW) == 0 and CH % 8 == 0
    assert n_ch % NBUF == 0 and CH <= 128 and T % NS == 0 and rpt % 8 == 0

    @functools.partial(
        pl.kernel,
        mesh=_sc_mesh(),
        out_type=jax.ShapeDtypeStruct((B, D), jnp.float32),
        scratch_types=[
            *[pltpu.VMEM((CH,), jnp.int32) for _ in range(2 * NBUF)],
            *[pltpu.VMEM((CH, D), jnp.float32) for _ in range(2 * NBUF)],
            *[pltpu.SemaphoreType.DMA for _ in range(3 * NBUF)],
        ],
    )
    def gather_add_k(ta_hbm, ia_hbm, tb_hbm, ib_hbm, out_hbm, *rest):
        iabufs = rest[:NBUF]
        ibbufs = rest[NBUF:2 * NBUF]
        bufs = rest[2 * NBUF:3 * NBUF]
        bufsb = rest[3 * NBUF:4 * NBUF]
        isem = rest[4 * NBUF:5 * NBUF]
        gsem = rest[5 * NBUF:6 * NBUF]
        osem = rest[6 * NBUF:]
        sid = lax.axis_index("s")
        wid = sid * NC + lax.axis_index("c")
        base = wid * b_per_w

        def start_idx(c, b):
            pltpu.async_copy(ia_hbm.at[pl.ds(base + c * CH, CH)], iabufs[b],
                             isem[b])
            pltpu.async_copy(ib_hbm.at[pl.ds(base + c * CH, CH)], ibbufs[b],
                             isem[b])

        def wait_idx(b):
            pltpu.make_async_copy(ia_hbm.at[pl.ds(base, CH)], iabufs[b],
                                  isem[b]).wait()
            pltpu.make_async_copy(ib_hbm.at[pl.ds(base, CH)], ibbufs[b],
                                  isem[b]).wait()

        def start_ga(b):
            pltpu.async_copy(ta_hbm.at[iabufs[b]], bufs[b], gsem[b])

        def wait_g(b):
            pltpu.make_async_copy(ta_hbm.at[iabufs[b]], bufs[b],
                                  gsem[b]).wait()

        def start_gb(b):
            pltpu.async_copy(tb_hbm.at[ibbufs[b]], bufsb[b], gsem[b])

        def accum(b):
            def arow(r, carry):
                for j in range(D // LANES):
                    sl = pl.ds(j * LANES, LANES)
                    bufs[b][r, sl] = bufs[b][r, sl] + bufsb[b][r, sl]
                return carry

            lax.fori_loop(0, CH, arow, 0)

        def start_out(c, b):
            pltpu.async_copy(bufs[b], out_hbm.at[pl.ds(base + c * CH, CH)],
                             osem[b])

        def wait_out(b):
            pltpu.make_async_copy(bufs[b], out_hbm.at[pl.ds(base, CH)],
                                  osem[b]).wait()

        for b in range(NBUF):
            start_idx(b, b)

        def body(g, carry):
            c0 = g * NBUF
            for b in range(NBUF):
                wait_idx(b)

                @pl.when(g > 0)
                def _():
                    wait_out(b)

                start_ga(b)
                start_gb(b)
            for b in range(NBUF):
                wait_g(b)
                wait_g(b)
                accum(b)
                start_out(c0 + b, b)

                @pl.when(g < n_grp - 1)
                def _():
                    start_idx(c0 + NBUF + b, b)

            return carry

        lax.fori_loop(0, n_grp, body, 0)
        for b in range(NBUF):
            wait_out(b)

    return gather_add_k


def _make_scatter_add(B, T, D, CH, NBUF):
    b_per_w = B // NW
    n_ch = b_per_w // CH
    n_grp = n_ch // NBUF
    rpt = T // NS
    assert b_per_w % CH == 0 and T % NS == 0 and CH % 8 == 0 and rpt % 8 == 0
    assert n_ch % NBUF == 0 and CH <= 128

    @functools.partial(
        pl.kernel,
        mesh=_sc_mesh(),
        out_type=jax.ShapeDtypeStruct((NC, T, D), jnp.float32),
        scratch_types=[
            *[pltpu.VMEM((CH,), jnp.int32) for _ in range(NBUF)],
            *[pltpu.VMEM((CH, D), jnp.float32) for _ in range(NBUF)],
            pltpu.VMEM_SHARED((T, D), jnp.float32),
            *[pltpu.SemaphoreType.DMA for _ in range(2 * NBUF)],
        ],
    )
    def scatter_k(vals_hbm, dest_hbm, out_hbm, *rest):
        ibufs = rest[:NBUF]
        bufs = rest[NBUF:2 * NBUF]
        acc_sh = rest[2 * NBUF]
        lsem = rest[2 * NBUF + 1:3 * NBUF + 1]
        asem = rest[3 * NBUF + 1:]
        cid = lax.axis_index("c")
        sid = lax.axis_index("s")
        wid = sid * NC + cid
        base = wid * b_per_w

        zero = jnp.zeros((LANES,), jnp.float32)
        zch = min(CH, rpt)
        zfull, zrem = divmod(rpt, zch)
        assert zrem % 8 == 0

        def zrow(r, carry):
            for j in range(D // LANES):
                bufs[0][r, pl.ds(j * LANES, LANES)] = zero
            return carry

        def zcopy(k, carry):
            pltpu.sync_copy(
                bufs[0].at[pl.ds(0, zch)],
                acc_sh.at[pl.ds(sid * rpt + k * zch, zch)],
            )
            return carry

        def start_load(c, b):
            pltpu.async_copy(vals_hbm.at[pl.ds(base + c * CH, CH)], bufs[b],
                             lsem[b])
            pltpu.async_copy(dest_hbm.at[pl.ds(base + c * CH, CH)], ibufs[b],
                             lsem[b])

        def wait_load(b):
            pltpu.make_async_copy(vals_hbm.at[pl.ds(base, CH)], bufs[b],
                                  lsem[b]).wait()
            pltpu.make_async_copy(dest_hbm.at[pl.ds(base, CH)], ibufs[b],
                                  lsem[b]).wait()

        def start_add(b):
            pltpu.async_copy(bufs[b], acc_sh.at[ibufs[b]], asem[b], add=True)

        def wait_add(b):
            pltpu.make_async_copy(bufs[b], acc_sh.at[ibufs[b]],
                                  asem[b]).wait()

        for b in range(1, NBUF):
            start_load(b, b)
        lax.fori_loop(0, zch, zrow, 0)
        lax.fori_loop(0, zfull, zcopy, 0)
        if zrem:
            pltpu.sync_copy(
                bufs[0].at[pl.ds(0, zrem)],
                acc_sh.at[pl.ds(sid * rpt + zfull * zch, zrem)],
            )
        start_load(0, 0)
        plsc.subcore_barrier()

        def body(g, carry):
            c0 = g * NBUF
            for b in range(NBUF):
                wait_load(b)
                start_add(b)
            for b in range(NBUF):
                wait_add(b)

                @pl.when(g < n_grp - 1)
                def _():
                    start_load(c0 + NBUF + b, b)

            return carry

        lax.fori_loop(0, n_grp, body, 0)
        plsc.subcore_barrier()
        pltpu.sync_copy(
            acc_sh.at[pl.ds(sid * rpt, rpt)],
            out_hbm.at[cid, pl.ds(sid * rpt, rpt)],
        )

    return scatter_k


def _proj_body(x_ref, w_ref, o_ref):
    o_ref[...] = jnp.dot(x_ref[...], w_ref[...],
                         preferred_element_type=jnp.float32)


def _init_body(asrc_ref, e_ref, wb_ref, w_ref, h0_ref, hw_ref):
    h0 = jnp.maximum(
        asrc_ref[...]
        + jnp.dot(e_ref[...], wb_ref[...], preferred_element_type=jnp.float32),
        0.0,
    )
    h0_ref[...] = h0
    hw_ref[...] = h0 * w_ref[...]


def _addp_body(p_ref, o_ref):
    o_ref[...] = p_ref[0] + p_ref[1]


def _addp_neg_body(p_ref, o_ref):
    o_ref[...] = -(p_ref[0] + p_ref[1])


def _combine_body(d_ref, h0_ref, w_ref, whn_ref, h_ref, hw_ref):
    h = jnp.maximum(
        jnp.dot(d_ref[...], whn_ref[...], preferred_element_type=jnp.float32)
        + h0_ref[...],
        0.0,
    )
    h_ref[...] = h
    hw_ref[...] = h * w_ref[...]


def _combine_last_body(d_ref, h0_ref, whn_ref, h_ref):
    h_ref[...] = jnp.maximum(
        jnp.dot(d_ref[...], whn_ref[...], preferred_element_type=jnp.float32)
        + h0_ref[...],
        0.0,
    )


def _pool_body(n_mols):
    def body(b_ref, h_ref, o_ref):
        i = pl.program_id(0)
        br = b_ref.shape[2]
        mol_ids = lax.broadcasted_iota(jnp.int32, (n_mols, br), 0)
        onehot = (mol_ids == b_ref[0]).astype(jnp.float32)
        r = jnp.dot(onehot, h_ref[...], preferred_element_type=jnp.float32)

        @pl.when(i == 0)
        def _():
            o_ref[...] = r

        @pl.when(i > 0)
        def _():
            o_ref[...] += r

    return body


def _atom_body(v_ref, mf_ref, woa_ref, wob_ref, b_ref, o_ref):
    m = mf_ref[0] + mf_ref[1]
    o_ref[...] = jnp.maximum(
        jnp.dot(v_ref[...], woa_ref[...], preferred_element_type=jnp.float32)
        + jnp.dot(m, wob_ref[...], preferred_element_type=jnp.float32)
        + b_ref[...],
        0.0,
    )


def _row_spec(br, d):
    return pl.BlockSpec((br, d), lambda i: (i, 0))


def _full_spec(shape):
    return pl.BlockSpec(shape, lambda i: tuple(0 for _ in shape))


def kernel(V, E, edge_index, rev_edge_index, batch, weight, W_i, W_h, W_o, b_o):
    n_atoms, atom_dim = V.shape
    n_edges, bond_dim = E.shape
    hidden = W_h.shape[0]

    src = edge_index[0]
    dest = edge_index[1]
    w2 = weight[:, None]
    wa = W_i[:atom_dim]
    wb = W_i[atom_dim:]
    woa = W_o[:atom_dim]
    wob = W_o[atom_dim:]
    b2 = b_o[None, :]

    BR_E = 2000
    BR_A = 2000
    CH = 80
    CHS = 40
    NBUF = 5

    t_acc = 10112
    gather = _make_gather(n_edges, hidden, CH, NBUF)
    scatter_edges = _make_scatter_add(n_edges, t_acc, hidden, CHS, NBUF)

    A = pl.pallas_call(
        _proj_body,
        grid=(n_atoms // BR_A,),
        in_specs=[_row_spec(BR_A, atom_dim), _full_spec((atom_dim, hidden))],
        out_specs=_row_spec(BR_A, hidden),
        out_shape=jax.ShapeDtypeStruct((n_atoms, hidden), jnp.float32),
    )(V, wa)

    a_src = gather(A, src)

    h0, hw = pl.pallas_call(
        _init_body,
        grid=(n_edges // BR_E,),
        in_specs=[
            _row_spec(BR_E, hidden),
            _row_spec(BR_E, bond_dim),
            _full_spec((bond_dim, hidden)),
            _row_spec(BR_E, 1),
        ],
        out_specs=[_row_spec(BR_E, hidden), _row_spec(BR_E, hidden)],
        out_shape=[
            jax.ShapeDtypeStruct((n_edges, hidden), jnp.float32),
            jax.ShapeDtypeStruct((n_edges, hidden), jnp.float32),
        ],
    )(a_src, E, wb, w2)

    add_partials_neg = pl.pallas_call(
        _addp_neg_body,
        grid=(t_acc // 1264,),
        in_specs=[pl.BlockSpec((NC, 1264, hidden), lambda i: (0, i, 0))],
        out_specs=_row_spec(1264, hidden),
        out_shape=jax.ShapeDtypeStruct((t_acc, hidden), jnp.float32),
    )

    combine = pl.pallas_call(
        _combine_body,
        grid=(n_edges // BR_E,),
        in_specs=[
            _row_spec(BR_E, hidden),
            _row_spec(BR_E, hidden),
            _row_spec(BR_E, 1),
            _full_spec((hidden, hidden)),
        ],
        out_specs=[_row_spec(BR_E, hidden), _row_spec(BR_E, hidden)],
        out_shape=[
            jax.ShapeDtypeStruct((n_edges, hidden), jnp.float32),
            jax.ShapeDtypeStruct((n_edges, hidden), jnp.float32),
        ],
    )

    combine_last = pl.pallas_call(
        _combine_last_body,
        grid=(n_edges // BR_E,),
        in_specs=[
            _row_spec(BR_E, hidden),
            _row_spec(BR_E, hidden),
            _full_spec((hidden, hidden)),
        ],
        out_specs=_row_spec(BR_E, hidden),
        out_shape=jax.ShapeDtypeStruct((n_edges, hidden), jnp.float32),
    )

    gather_add = _make_gather_add(n_edges, t_acc, hidden, CH, NBUF)
    wh_neg = -W_h

    h = h0
    for it in range(3):
        s_neg = add_partials_neg(scatter_edges(hw, dest))
        d = gather_add(s_neg, src, hw, rev_edge_index)
        if it < 2:
            h, hw = combine(d, h0, w2, wh_neg)
        else:
            h = combine_last(d, h0, wh_neg)

    mf = scatter_edges(h, dest)

    h_atom = pl.pallas_call(
        _atom_body,
        grid=(n_atoms // BR_A,),
        in_specs=[
            _row_spec(BR_A, atom_dim),
            pl.BlockSpec((NC, BR_A, hidden), lambda i: (0, i, 0)),
            _full_spec((atom_dim, hidden)),
            _full_spec((hidden, hidden)),
            pl.BlockSpec((1, hidden), lambda i: (0, 0)),
        ],
        out_specs=_row_spec(BR_A, hidden),
        out_shape=jax.ShapeDtypeStruct((n_atoms, hidden), jnp.float32),
    )(V, mf, woa, wob, b2)

    n_mols = 256
    b2d = batch.reshape(n_atoms // BR_A, 1, BR_A)
    mol_vecs = pl.pallas_call(
        _pool_body(n_mols),
        grid=(n_atoms // BR_A,),
        in_specs=[
            pl.BlockSpec((1, 1, BR_A), lambda i: (i, 0, 0)),
            _row_spec(BR_A, hidden),
        ],
        out_specs=pl.BlockSpec((n_mols, hidden), lambda i: (0, 0)),
        out_shape=jax.ShapeDtypeStruct((n_mols, hidden), jnp.float32),
    )(b2d, h_atom)

    return (h_atom, batch, mol_vecs, h)


# speedup vs baseline: 2.9332x; 1.0360x over previous
import functools

import jax
import jax.numpy as jnp
from jax import lax
from jax.experimental import pallas as pl
from jax.experimental.pallas import tpu as pltpu
from jax.experimental.pallas import tpu_sc as plsc

NC = 2
NS = 16
NW = NC * NS
LANES = 16

HIDDEN = 128


def _sc_mesh():
    return plsc.VectorSubcoreMesh(core_axis_name="c", subcore_axis_name="s")


def _make_gather(B, D, CH, NBUF):
    b_per_w = B // NW
    n_ch = b_per_w // CH
    n_grp = n_ch // NBUF
    assert b_per_w % CH == 0 and B % (8 * NW) == 0 and CH % 8 == 0
    assert n_ch % NBUF == 0 and CH <= 128

    @functools.partial(
        pl.kernel,
        mesh=_sc_mesh(),
        out_type=jax.ShapeDtypeStruct((B, D), jnp.float32),
        scratch_types=[
            *[pltpu.VMEM((CH,), jnp.int32) for _ in range(NBUF)],
            *[pltpu.VMEM((CH, D), jnp.float32) for _ in range(NBUF)],
            *[pltpu.SemaphoreType.DMA for _ in range(3 * NBUF)],
        ],
    )
    def gather_k(table_hbm, idx_hbm, out_hbm, *rest):
        ibufs = rest[:NBUF]
        bufs = rest[NBUF:2 * NBUF]
        isem = rest[2 * NBUF:3 * NBUF]
        gsem = rest[3 * NBUF:4 * NBUF]
        osem = rest[4 * NBUF:]
        wid = lax.axis_index("s") * NC + lax.axis_index("c")
        base = wid * b_per_w

        def start_idx(c, b):
            pltpu.async_copy(idx_hbm.at[pl.ds(base + c * CH, CH)], ibufs[b],
                             isem[b])

        def wait_idx(b):
            pltpu.make_async_copy(idx_hbm.at[pl.ds(base, CH)], ibufs[b],
                                  isem[b]).wait()

        def start_gather(b):
            pltpu.async_copy(table_hbm.at[ibufs[b]], bufs[b], gsem[b])

        def wait_gather(b):
            pltpu.make_async_copy(table_hbm.at[ibufs[b]], bufs[b],
                                  gsem[b]).wait()

        def start_out(c, b):
            pltpu.async_copy(bufs[b], out_hbm.at[pl.ds(base + c * CH, CH)],
                             osem[b])

        def wait_out(b):
            pltpu.make_async_copy(bufs[b], out_hbm.at[pl.ds(base, CH)],
                                  osem[b]).wait()

        for b in range(NBUF):
            start_idx(b, b)

        def body(g, carry):
            c0 = g * NBUF
            for b in range(NBUF):
                wait_idx(b)

                @pl.when(g > 0)
                def _():
                    wait_out(b)

                start_gather(b)
            for b in range(NBUF):
                wait_gather(b)
                start_out(c0 + b, b)

                @pl.when(g < n_grp - 1)
                def _():
                    start_idx(c0 + NBUF + b, b)

            return carry

        lax.fori_loop(0, n_grp, body, 0)
        for b in range(NBUF):
            wait_out(b)

    return gather_k


def _make_gather_add(B, T, D, CH, NBUF):
    b_per_w = B // NW
    n_ch = b_per_w // CH
    n_grp = n_ch // NBUF
    rpt = T // NS
    assert b_per_w % CH == 0 and B % (8 * NW) == 0 and CH % 8 == 0
    assert n_ch % NBUF == 0 and CH <= 128 and T % NS == 0 and rpt % 8 == 0

    @functools.partial(
        pl.kernel,
        mesh=_sc_mesh(),
        out_type=jax.ShapeDtypeStruct((B, D), jnp.float32),
        scratch_types=[
            *[pltpu.VMEM((CH,), jnp.int32) for _ in range(2 * NBUF)],
            *[pltpu.VMEM((CH, D), jnp.float32) for _ in range(2 * NBUF)],
            *[pltpu.SemaphoreType.DMA for _ in range(3 * NBUF)],
        ],
    )
    def gather_add_k(ta_hbm, ia_hbm, tb_hbm, ib_hbm, out_hbm, *rest):
        iabufs = rest[:NBUF]
        ibbufs = rest[NBUF:2 * NBUF]
        bufs = rest[2 * NBUF:3 * NBUF]
        bufsb = rest[3 * NBUF:4 * NBUF]
        isem = rest[4 * NBUF:5 * NBUF]
        gsem = rest[5 * NBUF:6 * NBUF]
        osem = rest[6 * NBUF:]
        sid = lax.axis_index("s")
        wid = sid * NC + lax.axis_index("c")
        base = wid * b_per_w

        def start_idx(c, b):
            pltpu.async_copy(ia_hbm.at[pl.ds(base + c * CH, CH)], iabufs[b],
                             isem[b])
            pltpu.async_copy(ib_hbm.at[pl.ds(base + c * CH, CH)], ibbufs[b],
                             isem[b])

        def wait_idx(b):
            pltpu.make_async_copy(ia_hbm.at[pl.ds(base, CH)], iabufs[b],
                                  isem[b]).wait()
            pltpu.make_async_copy(ib_hbm.at[pl.ds(base, CH)], ibbufs[b],
                                  isem[b]).wait()

        def start_ga(b):
            pltpu.async_copy(ta_hbm.at[iabufs[b]], bufs[b], gsem[b])

        def wait_g(b):
            pltpu.make_async_copy(ta_hbm.at[iabufs[b]], bufs[b],
                                  gsem[b]).wait()

        def start_gb(b):
            pltpu.async_copy(tb_hbm.at[ibbufs[b]], bufsb[b], gsem[b])

        def accum(b):
            def arow(r, carry):
                for j in range(D // LANES):
                    sl = pl.ds(j * LANES, LANES)
                    bufs[b][r, sl] = bufs[b][r, sl] + bufsb[b][r, sl]
                return carry

            lax.fori_loop(0, CH, arow, 0)

        def start_out(c, b):
            pltpu.async_copy(bufs[b], out_hbm.at[pl.ds(base + c * CH, CH)],
                             osem[b])

        def wait_out(b):
            pltpu.make_async_copy(bufs[b], out_hbm.at[pl.ds(base, CH)],
                                  osem[b]).wait()

        for b in range(NBUF):
            start_idx(b, b)

        def body(g, carry):
            c0 = g * NBUF
            for b in range(NBUF):
                wait_idx(b)

                @pl.when(g > 0)
                def _():
                    wait_out(b)

                start_ga(b)
                start_gb(b)
            for b in range(NBUF):
                wait_g(b)
                wait_g(b)
                accum(b)
                start_out(c0 + b, b)

                @pl.when(g < n_grp - 1)
                def _():
                    start_idx(c0 + NBUF + b, b)

            return carry

        lax.fori_loop(0, n_grp, body, 0)
        for b in range(NBUF):
            wait_out(b)

    return gather_add_k


def _make_scatter_add(B, T, D, CH, NBUF):
    b_per_w = B // NW
    n_ch = b_per_w // CH
    n_grp = n_ch // NBUF
    rpt = T // NS
    assert b_per_w % CH == 0 and T % NS == 0 and CH % 8 == 0 and rpt % 8 == 0
    assert n_ch % NBUF == 0 and CH <= 128

    @functools.partial(
        pl.kernel,
        mesh=_sc_mesh(),
        out_type=jax.ShapeDtypeStruct((NC, T, D), jnp.float32),
        scratch_types=[
            *[pltpu.VMEM((CH,), jnp.int32) for _ in range(NBUF)],
            *[pltpu.VMEM((CH, D), jnp.float32) for _ in range(NBUF)],
            pltpu.VMEM_SHARED((T, D), jnp.float32),
            *[pltpu.SemaphoreType.DMA for _ in range(2 * NBUF)],
        ],
    )
    def scatter_k(vals_hbm, dest_hbm, out_hbm, *rest):
        ibufs = rest[:NBUF]
        bufs = rest[NBUF:2 * NBUF]
        acc_sh = rest[2 * NBUF]
        lsem = rest[2 * NBUF + 1:3 * NBUF + 1]
        asem = rest[3 * NBUF + 1:]
        cid = lax.axis_index("c")
        sid = lax.axis_index("s")
        wid = sid * NC + cid
        base = wid * b_per_w

        zero = jnp.zeros((LANES,), jnp.float32)
        zch = min(CH, rpt)
        zfull, zrem = divmod(rpt, zch)
        assert zrem % 8 == 0

        def zrow(r, carry):
            for j in range(D // LANES):
                bufs[0][r, pl.ds(j * LANES, LANES)] = zero
            return carry

        def zcopy(k, carry):
            pltpu.sync_copy(
                bufs[0].at[pl.ds(0, zch)],
                acc_sh.at[pl.ds(sid * rpt + k * zch, zch)],
            )
            return carry

        def start_load(c, b):
            pltpu.async_copy(vals_hbm.at[pl.ds(base + c * CH, CH)], bufs[b],
                             lsem[b])
            pltpu.async_copy(dest_hbm.at[pl.ds(base + c * CH, CH)], ibufs[b],
                             lsem[b])

        def wait_load(b):
            pltpu.make_async_copy(vals_hbm.at[pl.ds(base, CH)], bufs[b],
                                  lsem[b]).wait()
            pltpu.make_async_copy(dest_hbm.at[pl.ds(base, CH)], ibufs[b],
                                  lsem[b]).wait()

        def start_add(b):
            pltpu.async_copy(bufs[b], acc_sh.at[ibufs[b]], asem[b], add=True)

        def wait_add(b):
            pltpu.make_async_copy(bufs[b], acc_sh.at[ibufs[b]],
                                  asem[b]).wait()

        for b in range(1, NBUF):
            start_load(b, b)
        lax.fori_loop(0, zch, zrow, 0)
        lax.fori_loop(0, zfull, zcopy, 0)
        if zrem:
            pltpu.sync_copy(
                bufs[0].at[pl.ds(0, zrem)],
                acc_sh.at[pl.ds(sid * rpt + zfull * zch, zrem)],
            )
        start_load(0, 0)
        plsc.subcore_barrier()

        def body(g, carry):
            c0 = g * NBUF
            for b in range(NBUF):
                wait_load(b)
                start_add(b)
            for b in range(NBUF):
                wait_add(b)

                @pl.when(g < n_grp - 1)
                def _():
                    start_load(c0 + NBUF + b, b)

            return carry

        lax.fori_loop(0, n_grp, body, 0)
        plsc.subcore_barrier()
        pltpu.sync_copy(
            acc_sh.at[pl.ds(sid * rpt, rpt)],
            out_hbm.at[cid, pl.ds(sid * rpt, rpt)],
        )

    return scatter_k


def _proj_body(x_ref, w_ref, o_ref):
    o_ref[...] = jnp.dot(x_ref[...], w_ref[...],
                         preferred_element_type=jnp.float32)


def _init_body(asrc_ref, e_ref, wb_ref, w_ref, h0_ref, hw_ref):
    h0 = jnp.maximum(
        asrc_ref[...]
        + jnp.dot(e_ref[...], wb_ref[...], preferred_element_type=jnp.float32),
        0.0,
    )
    h0_ref[...] = h0
    hw_ref[...] = h0 * w_ref[...]


def _addp_body(p_ref, o_ref):
    o_ref[...] = p_ref[0] + p_ref[1]


def _addp_neg_body(p_ref, o_ref):
    o_ref[...] = -(p_ref[0] + p_ref[1])


def _combine_body(d_ref, h0_ref, w_ref, whn_ref, hw_ref):
    h = jnp.maximum(
        jnp.dot(d_ref[...], whn_ref[...], preferred_element_type=jnp.float32)
        + h0_ref[...],
        0.0,
    )
    hw_ref[...] = h * w_ref[...]


def _combine_last_body(d_ref, h0_ref, whn_ref, h_ref):
    h_ref[...] = jnp.maximum(
        jnp.dot(d_ref[...], whn_ref[...], preferred_element_type=jnp.float32)
        + h0_ref[...],
        0.0,
    )


def _pool_body(n_mols):
    def body(b_ref, h_ref, o_ref):
        i = pl.program_id(0)
        br = b_ref.shape[2]
        mol_ids = lax.broadcasted_iota(jnp.int32, (n_mols, br), 0)
        onehot = (mol_ids == b_ref[0]).astype(jnp.float32)
        r = jnp.dot(onehot, h_ref[...], preferred_element_type=jnp.float32)

        @pl.when(i == 0)
        def _():
            o_ref[...] = r

        @pl.when(i > 0)
        def _():
            o_ref[...] += r

    return body


def _atom_body(v_ref, mf_ref, woa_ref, wob_ref, b_ref, o_ref):
    m = mf_ref[0] + mf_ref[1]
    o_ref[...] = jnp.maximum(
        jnp.dot(v_ref[...], woa_ref[...], preferred_element_type=jnp.float32)
        + jnp.dot(m, wob_ref[...], preferred_element_type=jnp.float32)
        + b_ref[...],
        0.0,
    )


def _row_spec(br, d):
    return pl.BlockSpec((br, d), lambda i: (i, 0))


def _full_spec(shape):
    return pl.BlockSpec(shape, lambda i: tuple(0 for _ in shape))


def kernel(V, E, edge_index, rev_edge_index, batch, weight, W_i, W_h, W_o, b_o):
    n_atoms, atom_dim = V.shape
    n_edges, bond_dim = E.shape
    hidden = W_h.shape[0]

    src = edge_index[0]
    dest = edge_index[1]
    w2 = weight[:, None]
    wa = W_i[:atom_dim]
    wb = W_i[atom_dim:]
    woa = W_o[:atom_dim]
    wob = W_o[atom_dim:]
    b2 = b_o[None, :]

    BR_E = 2000
    BR_A = 2000
    CH = 80
    CHS = 40
    NBUF = 5

    t_acc = 10112
    gather = _make_gather(n_edges, hidden, CH, NBUF)
    scatter_edges = _make_scatter_add(n_edges, t_acc, hidden, CHS, NBUF)

    A = pl.pallas_call(
        _proj_body,
        grid=(n_atoms // BR_A,),
        in_specs=[_row_spec(BR_A, atom_dim), _full_spec((atom_dim, hidden))],
        out_specs=_row_spec(BR_A, hidden),
        out_shape=jax.ShapeDtypeStruct((n_atoms, hidden), jnp.float32),
    )(V, wa)

    a_src = gather(A, src)

    h0, hw = pl.pallas_call(
        _init_body,
        grid=(n_edges // BR_E,),
        in_specs=[
            _row_spec(BR_E, hidden),
            _row_spec(BR_E, bond_dim),
            _full_spec((bond_dim, hidden)),
            _row_spec(BR_E, 1),
        ],
        out_specs=[_row_spec(BR_E, hidden), _row_spec(BR_E, hidden)],
        out_shape=[
            jax.ShapeDtypeStruct((n_edges, hidden), jnp.float32),
            jax.ShapeDtypeStruct((n_edges, hidden), jnp.float32),
        ],
    )(a_src, E, wb, w2)

    add_partials_neg = pl.pallas_call(
        _addp_neg_body,
        grid=(t_acc // 1264,),
        in_specs=[pl.BlockSpec((NC, 1264, hidden), lambda i: (0, i, 0))],
        out_specs=_row_spec(1264, hidden),
        out_shape=jax.ShapeDtypeStruct((t_acc, hidden), jnp.float32),
    )

    combine = pl.pallas_call(
        _combine_body,
        grid=(n_edges // BR_E,),
        in_specs=[
            _row_spec(BR_E, hidden),
            _row_spec(BR_E, hidden),
            _row_spec(BR_E, 1),
            _full_spec((hidden, hidden)),
        ],
        out_specs=_row_spec(BR_E, hidden),
        out_shape=jax.ShapeDtypeStruct((n_edges, hidden), jnp.float32),
    )

    combine_last = pl.pallas_call(
        _combine_last_body,
        grid=(n_edges // BR_E,),
        in_specs=[
            _row_spec(BR_E, hidden),
            _row_spec(BR_E, hidden),
            _full_spec((hidden, hidden)),
        ],
        out_specs=_row_spec(BR_E, hidden),
        out_shape=jax.ShapeDtypeStruct((n_edges, hidden), jnp.float32),
    )

    gather_add = _make_gather_add(n_edges, t_acc, hidden, CH, NBUF)
    wh_neg = -W_h

    for it in range(3):
        s_neg = add_partials_neg(scatter_edges(hw, dest))
        d = gather_add(s_neg, src, hw, rev_edge_index)
        if it < 2:
            hw = combine(d, h0, w2, wh_neg)
        else:
            h = combine_last(d, h0, wh_neg)

    mf = scatter_edges(h, dest)

    h_atom = pl.pallas_call(
        _atom_body,
        grid=(n_atoms // BR_A,),
        in_specs=[
            _row_spec(BR_A, atom_dim),
            pl.BlockSpec((NC, BR_A, hidden), lambda i: (0, i, 0)),
            _full_spec((atom_dim, hidden)),
            _full_spec((hidden, hidden)),
            pl.BlockSpec((1, hidden), lambda i: (0, 0)),
        ],
        out_specs=_row_spec(BR_A, hidden),
        out_shape=jax.ShapeDtypeStruct((n_atoms, hidden), jnp.float32),
    )(V, mf, woa, wob, b2)

    n_mols = 256
    b2d = batch.reshape(n_atoms // BR_A, 1, BR_A)
    mol_vecs = pl.pallas_call(
        _pool_body(n_mols),
        grid=(n_atoms // BR_A,),
        in_specs=[
            pl.BlockSpec((1, 1, BR_A), lambda i: (i, 0, 0)),
            _row_spec(BR_A, hidden),
        ],
        out_specs=pl.BlockSpec((n_mols, hidden), lambda i: (0, 0)),
        out_shape=jax.ShapeDtypeStruct((n_mols, hidden), jnp.float32),
    )(b2d, h_atom)

    return (h_atom, batch, mol_vecs, h)


# speedup vs baseline: 3.0936x; 1.0547x over previous
import functools

import jax
import jax.numpy as jnp
from jax import lax
from jax.experimental import pallas as pl
from jax.experimental.pallas import tpu as pltpu
from jax.experimental.pallas import tpu_sc as plsc

NC = 2
NS = 16
NW = NC * NS
LANES = 16

HIDDEN = 128


def _sc_mesh():
    return plsc.VectorSubcoreMesh(core_axis_name="c", subcore_axis_name="s")


def _make_gather(B, D, CH, NBUF):
    b_per_w = B // NW
    n_ch = b_per_w // CH
    n_grp = n_ch // NBUF
    assert b_per_w % CH == 0 and B % (8 * NW) == 0 and CH % 8 == 0
    assert n_ch % NBUF == 0 and CH <= 128

    @functools.partial(
        pl.kernel,
        mesh=_sc_mesh(),
        out_type=jax.ShapeDtypeStruct((B, D), jnp.float32),
        scratch_types=[
            *[pltpu.VMEM((CH,), jnp.int32) for _ in range(NBUF)],
            *[pltpu.VMEM((CH, D), jnp.float32) for _ in range(NBUF)],
            *[pltpu.SemaphoreType.DMA for _ in range(3 * NBUF)],
        ],
    )
    def gather_k(table_hbm, idx_hbm, out_hbm, *rest):
        ibufs = rest[:NBUF]
        bufs = rest[NBUF:2 * NBUF]
        isem = rest[2 * NBUF:3 * NBUF]
        gsem = rest[3 * NBUF:4 * NBUF]
        osem = rest[4 * NBUF:]
        wid = lax.axis_index("s") * NC + lax.axis_index("c")
        base = wid * b_per_w

        def start_idx(c, b):
            pltpu.async_copy(idx_hbm.at[pl.ds(base + c * CH, CH)], ibufs[b],
                             isem[b])

        def wait_idx(b):
            pltpu.make_async_copy(idx_hbm.at[pl.ds(base, CH)], ibufs[b],
                                  isem[b]).wait()

        def start_gather(b):
            pltpu.async_copy(table_hbm.at[ibufs[b]], bufs[b], gsem[b])

        def wait_gather(b):
            pltpu.make_async_copy(table_hbm.at[ibufs[b]], bufs[b],
                                  gsem[b]).wait()

        def start_out(c, b):
            pltpu.async_copy(bufs[b], out_hbm.at[pl.ds(base + c * CH, CH)],
                             osem[b])

        def wait_out(b):
            pltpu.make_async_copy(bufs[b], out_hbm.at[pl.ds(base, CH)],
                                  osem[b]).wait()

        for b in range(NBUF):
            start_idx(b, b)

        def body(g, carry):
            c0 = g * NBUF
            for b in range(NBUF):
                wait_idx(b)

                @pl.when(g > 0)
                def _():
                    wait_out(b)

                start_gather(b)
            for b in range(NBUF):
                wait_gather(b)
                start_out(c0 + b, b)

                @pl.when(g < n_grp - 1)
                def _():
                    start_idx(c0 + NBUF + b, b)

            return carry

        lax.fori_loop(0, n_grp, body, 0)
        for b in range(NBUF):
            wait_out(b)

    return gather_k


def _make_gather_add(B, T, D, CH, NBUF):
    b_per_w = B // NW
    n_ch = b_per_w // CH
    n_grp = n_ch // NBUF
    rpt = T // NS
    assert b_per_w % CH == 0 and B % (8 * NW) == 0 and CH % 8 == 0
    assert n_ch % NBUF == 0 and CH <= 128 and T % NS == 0 and rpt % 8 == 0

    @functools.partial(
        pl.kernel,
        mesh=_sc_mesh(),
        out_type=jax.ShapeDtypeStruct((B, D), jnp.float32),
        scratch_types=[
            *[pltpu.VMEM((CH,), jnp.int32) for _ in range(2 * NBUF)],
            *[pltpu.VMEM((CH, D), jnp.float32) for _ in range(2 * NBUF)],
            *[pltpu.SemaphoreType.DMA for _ in range(3 * NBUF)],
        ],
    )
    def gather_add_k(ta_hbm, ia_hbm, tb_hbm, ib_hbm, out_hbm, *rest):
        iabufs = rest[:NBUF]
        ibbufs = rest[NBUF:2 * NBUF]
        bufs = rest[2 * NBUF:3 * NBUF]
        bufsb = rest[3 * NBUF:4 * NBUF]
        isem = rest[4 * NBUF:5 * NBUF]
        gsem = rest[5 * NBUF:6 * NBUF]
        osem = rest[6 * NBUF:]
        sid = lax.axis_index("s")
        wid = sid * NC + lax.axis_index("c")
        base = wid * b_per_w

        def start_idx(c, b):
            pltpu.async_copy(ia_hbm.at[pl.ds(base + c * CH, CH)], iabufs[b],
                             isem[b])
            pltpu.async_copy(ib_hbm.at[pl.ds(base + c * CH, CH)], ibbufs[b],
                             isem[b])

        def wait_idx(b):
            pltpu.make_async_copy(ia_hbm.at[pl.ds(base, CH)], iabufs[b],
                                  isem[b]).wait()
            pltpu.make_async_copy(ib_hbm.at[pl.ds(base, CH)], ibbufs[b],
                                  isem[b]).wait()

        def start_ga(b):
            pltpu.async_copy(ta_hbm.at[iabufs[b]], bufs[b], gsem[b])

        def wait_g(b):
            pltpu.make_async_copy(ta_hbm.at[iabufs[b]], bufs[b],
                                  gsem[b]).wait()

        def start_gb(b):
            pltpu.async_copy(tb_hbm.at[ibbufs[b]], bufsb[b], gsem[b])

        def accum(b):
            def arow(r, carry):
                for j in range(D // LANES):
                    sl = pl.ds(j * LANES, LANES)
                    bufs[b][r, sl] = bufs[b][r, sl] + bufsb[b][r, sl]
                return carry

            lax.fori_loop(0, CH, arow, 0)

        def start_out(c, b):
            pltpu.async_copy(bufs[b], out_hbm.at[pl.ds(base + c * CH, CH)],
                             osem[b])

        def wait_out(b):
            pltpu.make_async_copy(bufs[b], out_hbm.at[pl.ds(base, CH)],
                                  osem[b]).wait()

        for b in range(NBUF):
            start_idx(b, b)

        def body(g, carry):
            c0 = g * NBUF
            for b in range(NBUF):
                wait_idx(b)

                @pl.when(g > 0)
                def _():
                    wait_out(b)

                start_ga(b)
                start_gb(b)
            for b in range(NBUF):
                wait_g(b)
                wait_g(b)
                accum(b)
                start_out(c0 + b, b)

                @pl.when(g < n_grp - 1)
                def _():
                    start_idx(c0 + NBUF + b, b)

            return carry

        lax.fori_loop(0, n_grp, body, 0)
        for b in range(NBUF):
            wait_out(b)

    return gather_add_k


def _make_scatter_add(B, T, D, CH, NBUF):
    b_per_w = B // NW
    n_ch = b_per_w // CH
    n_grp = n_ch // NBUF
    rpt = T // NS
    assert b_per_w % CH == 0 and T % NS == 0 and CH % 8 == 0 and rpt % 8 == 0
    assert n_ch % NBUF == 0 and CH <= 128

    @functools.partial(
        pl.kernel,
        mesh=_sc_mesh(),
        out_type=jax.ShapeDtypeStruct((NC, T, D), jnp.float32),
        scratch_types=[
            *[pltpu.VMEM((CH,), jnp.int32) for _ in range(NBUF)],
            *[pltpu.VMEM((CH, D), jnp.float32) for _ in range(NBUF)],
            pltpu.VMEM_SHARED((T, D), jnp.float32),
            *[pltpu.SemaphoreType.DMA for _ in range(2 * NBUF)],
        ],
    )
    def scatter_k(vals_hbm, dest_hbm, out_hbm, *rest):
        ibufs = rest[:NBUF]
        bufs = rest[NBUF:2 * NBUF]
        acc_sh = rest[2 * NBUF]
        lsem = rest[2 * NBUF + 1:3 * NBUF + 1]
        asem = rest[3 * NBUF + 1:]
        cid = lax.axis_index("c")
        sid = lax.axis_index("s")
        wid = sid * NC + cid
        base = wid * b_per_w

        zero = jnp.zeros((LANES,), jnp.float32)
        zch = min(CH, rpt)
        zfull, zrem = divmod(rpt, zch)
        assert zrem % 8 == 0

        def zrow(r, carry):
            for j in range(D // LANES):
                bufs[0][r, pl.ds(j * LANES, LANES)] = zero
            return carry

        def zcopy(k, carry):
            pltpu.sync_copy(
                bufs[0].at[pl.ds(0, zch)],
                acc_sh.at[pl.ds(sid * rpt + k * zch, zch)],
            )
            return carry

        def start_load(c, b):
            pltpu.async_copy(vals_hbm.at[pl.ds(base + c * CH, CH)], bufs[b],
                             lsem[b])
            pltpu.async_copy(dest_hbm.at[pl.ds(base + c * CH, CH)], ibufs[b],
                             lsem[b])

        def wait_load(b):
            pltpu.make_async_copy(vals_hbm.at[pl.ds(base, CH)], bufs[b],
                                  lsem[b]).wait()
            pltpu.make_async_copy(dest_hbm.at[pl.ds(base, CH)], ibufs[b],
                                  lsem[b]).wait()

        def start_add(b):
            pltpu.async_copy(bufs[b], acc_sh.at[ibufs[b]], asem[b], add=True)

        def wait_add(b):
            pltpu.make_async_copy(bufs[b], acc_sh.at[ibufs[b]],
                                  asem[b]).wait()

        for b in range(1, NBUF):
            start_load(b, b)
        lax.fori_loop(0, zch, zrow, 0)
        lax.fori_loop(0, zfull, zcopy, 0)
        if zrem:
            pltpu.sync_copy(
                bufs[0].at[pl.ds(0, zrem)],
                acc_sh.at[pl.ds(sid * rpt + zfull * zch, zrem)],
            )
        start_load(0, 0)
        plsc.subcore_barrier()

        def body(g, carry):
            c0 = g * NBUF
            for b in range(NBUF):
                wait_load(b)
                start_add(b)
            for b in range(NBUF):
                wait_add(b)

                @pl.when(g < n_grp - 1)
                def _():
                    start_load(c0 + NBUF + b, b)

            return carry

        lax.fori_loop(0, n_grp, body, 0)
        plsc.subcore_barrier()
        pltpu.sync_copy(
            acc_sh.at[pl.ds(sid * rpt, rpt)],
            out_hbm.at[cid, pl.ds(sid * rpt, rpt)],
        )

    return scatter_k


def _proj_body(x_ref, w_ref, o_ref):
    o_ref[...] = jnp.dot(x_ref[...], w_ref[...],
                         preferred_element_type=jnp.float32)


def _init_body(asrc_ref, e_ref, wb_ref, w_ref, h0_ref, hw_ref):
    h0 = jnp.maximum(
        asrc_ref[...]
        + jnp.dot(e_ref[...], wb_ref[...], preferred_element_type=jnp.float32),
        0.0,
    )
    h0_ref[...] = h0
    hw_ref[...] = h0 * w_ref[...]


def _addp_body(p_ref, o_ref):
    o_ref[...] = p_ref[0] + p_ref[1]


def _addp_neg_body(p_ref, o_ref):
    o_ref[...] = -(p_ref[0] + p_ref[1])


def _combine_body(d_ref, h0_ref, w_ref, whn_ref, hw_ref):
    h = jnp.maximum(
        jnp.dot(d_ref[...], whn_ref[...], preferred_element_type=jnp.float32)
        + h0_ref[...],
        0.0,
    )
    hw_ref[...] = h * w_ref[...]


def _combine_last_body(d_ref, h0_ref, whn_ref, h_ref):
    h_ref[...] = jnp.maximum(
        jnp.dot(d_ref[...], whn_ref[...], preferred_element_type=jnp.float32)
        + h0_ref[...],
        0.0,
    )


def _pool_body(n_mols):
    def body(b_ref, h_ref, o_ref):
        i = pl.program_id(0)
        br = b_ref.shape[2]
        mol_ids = lax.broadcasted_iota(jnp.int32, (n_mols, br), 0)
        onehot = (mol_ids == b_ref[0]).astype(jnp.float32)
        r = jnp.dot(onehot, h_ref[...], preferred_element_type=jnp.float32)

        @pl.when(i == 0)
        def _():
            o_ref[...] = r

        @pl.when(i > 0)
        def _():
            o_ref[...] += r

    return body


def _atom_body(v_ref, mf_ref, woa_ref, wob_ref, b_ref, o_ref):
    m = mf_ref[0] + mf_ref[1]
    o_ref[...] = jnp.maximum(
        jnp.dot(v_ref[...], woa_ref[...], preferred_element_type=jnp.float32)
        + jnp.dot(m, wob_ref[...], preferred_element_type=jnp.float32)
        + b_ref[...],
        0.0,
    )


def _row_spec(br, d):
    return pl.BlockSpec((br, d), lambda i: (i, 0))


def _full_spec(shape):
    return pl.BlockSpec(shape, lambda i: tuple(0 for _ in shape))


def kernel(V, E, edge_index, rev_edge_index, batch, weight, W_i, W_h, W_o, b_o):
    n_atoms, atom_dim = V.shape
    n_edges, bond_dim = E.shape
    hidden = W_h.shape[0]

    src = edge_index[0]
    dest = edge_index[1]
    w2 = weight[:, None]
    wa = W_i[:atom_dim]
    wb = W_i[atom_dim:]
    woa = W_o[:atom_dim]
    wob = W_o[atom_dim:]
    b2 = b_o[None, :]

    BR_E = 4000
    BR_A = 2000
    CH = 80
    CHS = 40
    NBUF = 5

    t_acc = 10112
    gather = _make_gather(n_edges, hidden, CH, NBUF)
    scatter_edges = _make_scatter_add(n_edges, t_acc, hidden, CHS, NBUF)

    A = pl.pallas_call(
        _proj_body,
        grid=(n_atoms // BR_A,),
        in_specs=[_row_spec(BR_A, atom_dim), _full_spec((atom_dim, hidden))],
        out_specs=_row_spec(BR_A, hidden),
        out_shape=jax.ShapeDtypeStruct((n_atoms, hidden), jnp.float32),
    )(V, wa)

    a_src = gather(A, src)

    h0, hw = pl.pallas_call(
        _init_body,
        grid=(n_edges // BR_E,),
        in_specs=[
            _row_spec(BR_E, hidden),
            _row_spec(BR_E, bond_dim),
            _full_spec((bond_dim, hidden)),
            _row_spec(BR_E, 1),
        ],
        out_specs=[_row_spec(BR_E, hidden), _row_spec(BR_E, hidden)],
        out_shape=[
            jax.ShapeDtypeStruct((n_edges, hidden), jnp.float32),
            jax.ShapeDtypeStruct((n_edges, hidden), jnp.float32),
        ],
    )(a_src, E, wb, w2)

    add_partials_neg = pl.pallas_call(
        _addp_neg_body,
        grid=(t_acc // 1264,),
        in_specs=[pl.BlockSpec((NC, 1264, hidden), lambda i: (0, i, 0))],
        out_specs=_row_spec(1264, hidden),
        out_shape=jax.ShapeDtypeStruct((t_acc, hidden), jnp.float32),
    )

    combine = pl.pallas_call(
        _combine_body,
        grid=(n_edges // BR_E,),
        in_specs=[
            _row_spec(BR_E, hidden),
            _row_spec(BR_E, hidden),
            _row_spec(BR_E, 1),
            _full_spec((hidden, hidden)),
        ],
        out_specs=_row_spec(BR_E, hidden),
        out_shape=jax.ShapeDtypeStruct((n_edges, hidden), jnp.float32),
    )

    combine_last = pl.pallas_call(
        _combine_last_body,
        grid=(n_edges // BR_E,),
        in_specs=[
            _row_spec(BR_E, hidden),
            _row_spec(BR_E, hidden),
            _full_spec((hidden, hidden)),
        ],
        out_specs=_row_spec(BR_E, hidden),
        out_shape=jax.ShapeDtypeStruct((n_edges, hidden), jnp.float32),
    )

    gather_add = _make_gather_add(n_edges, t_acc, hidden, CH, NBUF)
    wh_neg = -W_h

    for it in range(3):
        s_neg = add_partials_neg(scatter_edges(hw, dest))
        d = gather_add(s_neg, src, hw, rev_edge_index)
        if it < 2:
            hw = combine(d, h0, w2, wh_neg)
        else:
            h = combine_last(d, h0, wh_neg)

    mf = scatter_edges(h, dest)

    h_atom = pl.pallas_call(
        _atom_body,
        grid=(n_atoms // BR_A,),
        in_specs=[
            _row_spec(BR_A, atom_dim),
            pl.BlockSpec((NC, BR_A, hidden), lambda i: (0, i, 0)),
            _full_spec((atom_dim, hidden)),
            _full_spec((hidden, hidden)),
            pl.BlockSpec((1, hidden), lambda i: (0, 0)),
        ],
        out_specs=_row_spec(BR_A, hidden),
        out_shape=jax.ShapeDtypeStruct((n_atoms, hidden), jnp.float32),
    )(V, mf, woa, wob, b2)

    n_mols = 256
    b2d = batch.reshape(n_atoms // BR_A, 1, BR_A)
    mol_vecs = pl.pallas_call(
        _pool_body(n_mols),
        grid=(n_atoms // BR_A,),
        in_specs=[
            pl.BlockSpec((1, 1, BR_A), lambda i: (i, 0, 0)),
            _row_spec(BR_A, hidden),
        ],
        out_specs=pl.BlockSpec((n_mols, hidden), lambda i: (0, 0)),
        out_shape=jax.ShapeDtypeStruct((n_mols, hidden), jnp.float32),
    )(b2d, h_atom)

    return (h_atom, batch, mol_vecs, h)


# speedup vs baseline: 3.1046x; 1.0036x over previous
import functools

import jax
import jax.numpy as jnp
from jax import lax
from jax.experimental import pallas as pl
from jax.experimental.pallas import tpu as pltpu
from jax.experimental.pallas import tpu_sc as plsc

NC = 2
NS = 16
NW = NC * NS
LANES = 16

HIDDEN = 128


def _sc_mesh():
    return plsc.VectorSubcoreMesh(core_axis_name="c", subcore_axis_name="s")


def _make_gather(B, D, CH, NBUF):
    b_per_w = B // NW
    n_ch = b_per_w // CH
    n_grp = n_ch // NBUF
    assert b_per_w % CH == 0 and B % (8 * NW) == 0 and CH % 8 == 0
    assert n_ch % NBUF == 0 and CH <= 128

    @functools.partial(
        pl.kernel,
        mesh=_sc_mesh(),
        out_type=jax.ShapeDtypeStruct((B, D), jnp.float32),
        scratch_types=[
            *[pltpu.VMEM((CH,), jnp.int32) for _ in range(NBUF)],
            *[pltpu.VMEM((CH, D), jnp.float32) for _ in range(NBUF)],
            *[pltpu.SemaphoreType.DMA for _ in range(3 * NBUF)],
        ],
    )
    def gather_k(table_hbm, idx_hbm, out_hbm, *rest):
        ibufs = rest[:NBUF]
        bufs = rest[NBUF:2 * NBUF]
        isem = rest[2 * NBUF:3 * NBUF]
        gsem = rest[3 * NBUF:4 * NBUF]
        osem = rest[4 * NBUF:]
        wid = lax.axis_index("s") * NC + lax.axis_index("c")
        base = wid * b_per_w

        def start_idx(c, b):
            pltpu.async_copy(idx_hbm.at[pl.ds(base + c * CH, CH)], ibufs[b],
                             isem[b])

        def wait_idx(b):
            pltpu.make_async_copy(idx_hbm.at[pl.ds(base, CH)], ibufs[b],
                                  isem[b]).wait()

        def start_gather(b):
            pltpu.async_copy(table_hbm.at[ibufs[b]], bufs[b], gsem[b])

        def wait_gather(b):
            pltpu.make_async_copy(table_hbm.at[ibufs[b]], bufs[b],
                                  gsem[b]).wait()

        def start_out(c, b):
            pltpu.async_copy(bufs[b], out_hbm.at[pl.ds(base + c * CH, CH)],
                             osem[b])

        def wait_out(b):
            pltpu.make_async_copy(bufs[b], out_hbm.at[pl.ds(base, CH)],
                                  osem[b]).wait()

        for b in range(NBUF):
            start_idx(b, b)

        def body(g, carry):
            c0 = g * NBUF
            for b in range(NBUF):
                wait_idx(b)

                @pl.when(g > 0)
                def _():
                    wait_out(b)

                start_gather(b)
            for b in range(NBUF):
                wait_gather(b)
                start_out(c0 + b, b)

                @pl.when(g < n_grp - 1)
                def _():
                    start_idx(c0 + NBUF + b, b)

            return carry

        lax.fori_loop(0, n_grp, body, 0)
        for b in range(NBUF):
            wait_out(b)

    return gather_k


def _make_gather_add(B, T, D, CH, NBUF):
    b_per_w = B // NW
    n_ch = b_per_w // CH
    n_grp = n_ch // NBUF
    rpt = T // NS
    assert b_per_w % CH == 0 and B % (8 * NW) == 0 and CH % 8 == 0
    assert n_ch % NBUF == 0 and CH <= 128 and T % NS == 0 and rpt % 8 == 0

    @functools.partial(
        pl.kernel,
        mesh=_sc_mesh(),
        out_type=jax.ShapeDtypeStruct((B, D), jnp.float32),
        scratch_types=[
            *[pltpu.VMEM((CH,), jnp.int32) for _ in range(2 * NBUF)],
            *[pltpu.VMEM((CH, D), jnp.float32) for _ in range(2 * NBUF)],
            *[pltpu.SemaphoreType.DMA for _ in range(3 * NBUF)],
        ],
    )
    def gather_add_k(ta_hbm, ia_hbm, tb_hbm, ib_hbm, out_hbm, *rest):
        iabufs = rest[:NBUF]
        ibbufs = rest[NBUF:2 * NBUF]
        bufs = rest[2 * NBUF:3 * NBUF]
        bufsb = rest[3 * NBUF:4 * NBUF]
        isem = rest[4 * NBUF:5 * NBUF]
        gsem = rest[5 * NBUF:6 * NBUF]
        osem = rest[6 * NBUF:]
        sid = lax.axis_index("s")
        wid = sid * NC + lax.axis_index("c")
        base = wid * b_per_w

        def start_idx(c, b):
            pltpu.async_copy(ia_hbm.at[pl.ds(base + c * CH, CH)], iabufs[b],
                             isem[b])
            pltpu.async_copy(ib_hbm.at[pl.ds(base + c * CH, CH)], ibbufs[b],
                             isem[b])

        def wait_idx(b):
            pltpu.make_async_copy(ia_hbm.at[pl.ds(base, CH)], iabufs[b],
                                  isem[b]).wait()
            pltpu.make_async_copy(ib_hbm.at[pl.ds(base, CH)], ibbufs[b],
                                  isem[b]).wait()

        def start_ga(b):
            pltpu.async_copy(ta_hbm.at[iabufs[b]], bufs[b], gsem[b])

        def wait_g(b):
            pltpu.make_async_copy(ta_hbm.at[iabufs[b]], bufs[b],
                                  gsem[b]).wait()

        def start_gb(b):
            pltpu.async_copy(tb_hbm.at[ibbufs[b]], bufsb[b], gsem[b])

        def accum(b):
            def arow(r, carry):
                for j in range(D // LANES):
                    sl = pl.ds(j * LANES, LANES)
                    bufs[b][r, sl] = bufs[b][r, sl] + bufsb[b][r, sl]
                return carry

            lax.fori_loop(0, CH, arow, 0)

        def start_out(c, b):
            pltpu.async_copy(bufs[b], out_hbm.at[pl.ds(base + c * CH, CH)],
                             osem[b])

        def wait_out(b):
            pltpu.make_async_copy(bufs[b], out_hbm.at[pl.ds(base, CH)],
                                  osem[b]).wait()

        for b in range(NBUF):
            start_idx(b, b)

        def body(g, carry):
            c0 = g * NBUF
            for b in range(NBUF):
                wait_idx(b)

                @pl.when(g > 0)
                def _():
                    wait_out(b)

                start_ga(b)
                start_gb(b)
            for b in range(NBUF):
                wait_g(b)
                wait_g(b)
                accum(b)
                start_out(c0 + b, b)

                @pl.when(g < n_grp - 1)
                def _():
                    start_idx(c0 + NBUF + b, b)

            return carry

        lax.fori_loop(0, n_grp, body, 0)
        for b in range(NBUF):
            wait_out(b)

    return gather_add_k


def _make_scatter_add(B, T, D, CH, NBUF):
    b_per_w = B // NW
    n_ch = b_per_w // CH
    n_grp = n_ch // NBUF
    rpt = T // NS
    assert b_per_w % CH == 0 and T % NS == 0 and CH % 8 == 0 and rpt % 8 == 0
    assert n_ch % NBUF == 0 and CH <= 128

    @functools.partial(
        pl.kernel,
        mesh=_sc_mesh(),
        out_type=jax.ShapeDtypeStruct((NC, T, D), jnp.float32),
        scratch_types=[
            *[pltpu.VMEM((CH,), jnp.int32) for _ in range(NBUF)],
            *[pltpu.VMEM((CH, D), jnp.float32) for _ in range(NBUF)],
            pltpu.VMEM_SHARED((T, D), jnp.float32),
            *[pltpu.SemaphoreType.DMA for _ in range(2 * NBUF)],
        ],
    )
    def scatter_k(vals_hbm, dest_hbm, out_hbm, *rest):
        ibufs = rest[:NBUF]
        bufs = rest[NBUF:2 * NBUF]
        acc_sh = rest[2 * NBUF]
        lsem = rest[2 * NBUF + 1:3 * NBUF + 1]
        asem = rest[3 * NBUF + 1:]
        cid = lax.axis_index("c")
        sid = lax.axis_index("s")
        wid = sid * NC + cid
        base = wid * b_per_w

        zero = jnp.zeros((LANES,), jnp.float32)
        zch = min(CH, rpt)
        zfull, zrem = divmod(rpt, zch)
        assert zrem % 8 == 0

        def zrow(r, carry):
            for j in range(D // LANES):
                bufs[0][r, pl.ds(j * LANES, LANES)] = zero
            return carry

        def zcopy(k, carry):
            pltpu.sync_copy(
                bufs[0].at[pl.ds(0, zch)],
                acc_sh.at[pl.ds(sid * rpt + k * zch, zch)],
            )
            return carry

        def start_load(c, b):
            pltpu.async_copy(vals_hbm.at[pl.ds(base + c * CH, CH)], bufs[b],
                             lsem[b])
            pltpu.async_copy(dest_hbm.at[pl.ds(base + c * CH, CH)], ibufs[b],
                             lsem[b])

        def wait_load(b):
            pltpu.make_async_copy(vals_hbm.at[pl.ds(base, CH)], bufs[b],
                                  lsem[b]).wait()
            pltpu.make_async_copy(dest_hbm.at[pl.ds(base, CH)], ibufs[b],
                                  lsem[b]).wait()

        def start_add(b):
            pltpu.async_copy(bufs[b], acc_sh.at[ibufs[b]], asem[b], add=True)

        def wait_add(b):
            pltpu.make_async_copy(bufs[b], acc_sh.at[ibufs[b]],
                                  asem[b]).wait()

        for b in range(1, NBUF):
            start_load(b, b)
        lax.fori_loop(0, zch, zrow, 0)
        lax.fori_loop(0, zfull, zcopy, 0)
        if zrem:
            pltpu.sync_copy(
                bufs[0].at[pl.ds(0, zrem)],
                acc_sh.at[pl.ds(sid * rpt + zfull * zch, zrem)],
            )
        start_load(0, 0)
        plsc.subcore_barrier()

        def body(g, carry):
            c0 = g * NBUF
            for b in range(NBUF):
                wait_load(b)
                start_add(b)
            for b in range(NBUF):
                wait_add(b)

                @pl.when(g < n_grp - 1)
                def _():
                    start_load(c0 + NBUF + b, b)

            return carry

        lax.fori_loop(0, n_grp, body, 0)
        plsc.subcore_barrier()
        pltpu.sync_copy(
            acc_sh.at[pl.ds(sid * rpt, rpt)],
            out_hbm.at[cid, pl.ds(sid * rpt, rpt)],
        )

    return scatter_k


def _proj_body(x_ref, w_ref, o_ref):
    o_ref[...] = jnp.dot(x_ref[...], w_ref[...],
                         preferred_element_type=jnp.float32)


def _init_body(asrc_ref, e_ref, wb_ref, w_ref, h0_ref, hw_ref):
    h0 = jnp.maximum(
        asrc_ref[...]
        + jnp.dot(e_ref[...], wb_ref[...], preferred_element_type=jnp.float32),
        0.0,
    )
    h0_ref[...] = h0
    hw_ref[...] = h0 * w_ref[...]


def _addp_body(p_ref, o_ref):
    o_ref[...] = p_ref[0] + p_ref[1]


def _addp_neg_body(p_ref, o_ref):
    o_ref[...] = -(p_ref[0] + p_ref[1])


def _combine_body(d_ref, h0_ref, w_ref, whn_ref, hw_ref):
    h = jnp.maximum(
        jnp.dot(d_ref[...], whn_ref[...], preferred_element_type=jnp.float32)
        + h0_ref[...],
        0.0,
    )
    hw_ref[...] = h * w_ref[...]


def _combine_last_body(d_ref, h0_ref, whn_ref, h_ref):
    h_ref[...] = jnp.maximum(
        jnp.dot(d_ref[...], whn_ref[...], preferred_element_type=jnp.float32)
        + h0_ref[...],
        0.0,
    )


def _pool_body(n_mols):
    def body(b_ref, h_ref, o_ref):
        i = pl.program_id(0)
        br = b_ref.shape[2]
        mol_ids = lax.broadcasted_iota(jnp.int32, (n_mols, br), 0)
        onehot = (mol_ids == b_ref[0]).astype(jnp.float32)
        r = jnp.dot(onehot, h_ref[...], preferred_element_type=jnp.float32)

        @pl.when(i == 0)
        def _():
            o_ref[...] = r

        @pl.when(i > 0)
        def _():
            o_ref[...] += r

    return body


def _atom_body(v_ref, mf_ref, woa_ref, wob_ref, b_ref, o_ref):
    m = mf_ref[0] + mf_ref[1]
    o_ref[...] = jnp.maximum(
        jnp.dot(v_ref[...], woa_ref[...], preferred_element_type=jnp.float32)
        + jnp.dot(m, wob_ref[...], preferred_element_type=jnp.float32)
        + b_ref[...],
        0.0,
    )


def _row_spec(br, d):
    return pl.BlockSpec((br, d), lambda i: (i, 0))


def _full_spec(shape):
    return pl.BlockSpec(shape, lambda i: tuple(0 for _ in shape))


def kernel(V, E, edge_index, rev_edge_index, batch, weight, W_i, W_h, W_o, b_o):
    n_atoms, atom_dim = V.shape
    n_edges, bond_dim = E.shape
    hidden = W_h.shape[0]

    src = edge_index[0]
    dest = edge_index[1]
    w2 = weight[:, None]
    wa = W_i[:atom_dim]
    wb = W_i[atom_dim:]
    woa = W_o[:atom_dim]
    wob = W_o[atom_dim:]
    b2 = b_o[None, :]

    BR_E = 8000
    BR_A = 2000
    CH = 80
    CHS = 40
    NBUF = 5

    t_acc = 10112
    gather = _make_gather(n_edges, hidden, CH, NBUF)
    scatter_edges = _make_scatter_add(n_edges, t_acc, hidden, CHS, NBUF)

    A = pl.pallas_call(
        _proj_body,
        grid=(n_atoms // BR_A,),
        in_specs=[_row_spec(BR_A, atom_dim), _full_spec((atom_dim, hidden))],
        out_specs=_row_spec(BR_A, hidden),
        out_shape=jax.ShapeDtypeStruct((n_atoms, hidden), jnp.float32),
    )(V, wa)

    a_src = gather(A, src)

    h0, hw = pl.pallas_call(
        _init_body,
        grid=(n_edges // BR_E,),
        in_specs=[
            _row_spec(BR_E, hidden),
            _row_spec(BR_E, bond_dim),
            _full_spec((bond_dim, hidden)),
            _row_spec(BR_E, 1),
        ],
        out_specs=[_row_spec(BR_E, hidden), _row_spec(BR_E, hidden)],
        out_shape=[
            jax.ShapeDtypeStruct((n_edges, hidden), jnp.float32),
            jax.ShapeDtypeStruct((n_edges, hidden), jnp.float32),
        ],
    )(a_src, E, wb, w2)

    add_partials_neg = pl.pallas_call(
        _addp_neg_body,
        grid=(t_acc // 1264,),
        in_specs=[pl.BlockSpec((NC, 1264, hidden), lambda i: (0, i, 0))],
        out_specs=_row_spec(1264, hidden),
        out_shape=jax.ShapeDtypeStruct((t_acc, hidden), jnp.float32),
    )

    combine = pl.pallas_call(
        _combine_body,
        grid=(n_edges // BR_E,),
        in_specs=[
            _row_spec(BR_E, hidden),
            _row_spec(BR_E, hidden),
            _row_spec(BR_E, 1),
            _full_spec((hidden, hidden)),
        ],
        out_specs=_row_spec(BR_E, hidden),
        out_shape=jax.ShapeDtypeStruct((n_edges, hidden), jnp.float32),
    )

    combine_last = pl.pallas_call(
        _combine_last_body,
        grid=(n_edges // BR_E,),
        in_specs=[
            _row_spec(BR_E, hidden),
            _row_spec(BR_E, hidden),
            _full_spec((hidden, hidden)),
        ],
        out_specs=_row_spec(BR_E, hidden),
        out_shape=jax.ShapeDtypeStruct((n_edges, hidden), jnp.float32),
    )

    gather_add = _make_gather_add(n_edges, t_acc, hidden, CH, NBUF)
    wh_neg = -W_h

    for it in range(3):
        s_neg = add_partials_neg(scatter_edges(hw, dest))
        d = gather_add(s_neg, src, hw, rev_edge_index)
        if it < 2:
            hw = combine(d, h0, w2, wh_neg)
        else:
            h = combine_last(d, h0, wh_neg)

    mf = scatter_edges(h, dest)

    h_atom = pl.pallas_call(
        _atom_body,
        grid=(n_atoms // BR_A,),
        in_specs=[
            _row_spec(BR_A, atom_dim),
            pl.BlockSpec((NC, BR_A, hidden), lambda i: (0, i, 0)),
            _full_spec((atom_dim, hidden)),
            _full_spec((hidden, hidden)),
            pl.BlockSpec((1, hidden), lambda i: (0, 0)),
        ],
        out_specs=_row_spec(BR_A, hidden),
        out_shape=jax.ShapeDtypeStruct((n_atoms, hidden), jnp.float32),
    )(V, mf, woa, wob, b2)

    n_mols = 256
    b2d = batch.reshape(n_atoms // BR_A, 1, BR_A)
    mol_vecs = pl.pallas_call(
        _pool_body(n_mols),
        grid=(n_atoms // BR_A,),
        in_specs=[
            pl.BlockSpec((1, 1, BR_A), lambda i: (i, 0, 0)),
            _row_spec(BR_A, hidden),
        ],
        out_specs=pl.BlockSpec((n_mols, hidden), lambda i: (0, 0)),
        out_shape=jax.ShapeDtypeStruct((n_mols, hidden), jnp.float32),
    )(b2d, h_atom)

    return (h_atom, batch, mol_vecs, h)


# speedup vs baseline: 3.2569x; 1.0490x over previous
import functools

import jax
import jax.numpy as jnp
from jax import lax
from jax.experimental import pallas as pl
from jax.experimental.pallas import tpu as pltpu
from jax.experimental.pallas import tpu_sc as plsc

NC = 2
NS = 16
NW = NC * NS
LANES = 16

HIDDEN = 128


def _sc_mesh():
    return plsc.VectorSubcoreMesh(core_axis_name="c", subcore_axis_name="s")


def _make_gather(B, D, CH, NBUF):
    b_per_w = B // NW
    n_ch = b_per_w // CH
    n_grp = n_ch // NBUF
    assert b_per_w % CH == 0 and B % (8 * NW) == 0 and CH % 8 == 0
    assert n_ch % NBUF == 0 and CH <= 128

    @functools.partial(
        pl.kernel,
        mesh=_sc_mesh(),
        out_type=jax.ShapeDtypeStruct((B, D), jnp.float32),
        scratch_types=[
            *[pltpu.VMEM((CH,), jnp.int32) for _ in range(NBUF)],
            *[pltpu.VMEM((CH, D), jnp.float32) for _ in range(NBUF)],
            *[pltpu.SemaphoreType.DMA for _ in range(3 * NBUF)],
        ],
    )
    def gather_k(table_hbm, idx_hbm, out_hbm, *rest):
        ibufs = rest[:NBUF]
        bufs = rest[NBUF:2 * NBUF]
        isem = rest[2 * NBUF:3 * NBUF]
        gsem = rest[3 * NBUF:4 * NBUF]
        osem = rest[4 * NBUF:]
        wid = lax.axis_index("s") * NC + lax.axis_index("c")
        base = wid * b_per_w

        def start_idx(c, b):
            pltpu.async_copy(idx_hbm.at[pl.ds(base + c * CH, CH)], ibufs[b],
                             isem[b])

        def wait_idx(b):
            pltpu.make_async_copy(idx_hbm.at[pl.ds(base, CH)], ibufs[b],
                                  isem[b]).wait()

        def start_gather(b):
            pltpu.async_copy(table_hbm.at[ibufs[b]], bufs[b], gsem[b])

        def wait_gather(b):
            pltpu.make_async_copy(table_hbm.at[ibufs[b]], bufs[b],
                                  gsem[b]).wait()

        def start_out(c, b):
            pltpu.async_copy(bufs[b], out_hbm.at[pl.ds(base + c * CH, CH)],
                             osem[b])

        def wait_out(b):
            pltpu.make_async_copy(bufs[b], out_hbm.at[pl.ds(base, CH)],
                                  osem[b]).wait()

        for b in range(NBUF):
            start_idx(b, b)

        def body(g, carry):
            c0 = g * NBUF
            for b in range(NBUF):
                wait_idx(b)

                @pl.when(g > 0)
                def _():
                    wait_out(b)

                start_gather(b)
            for b in range(NBUF):
                wait_gather(b)
                start_out(c0 + b, b)

                @pl.when(g < n_grp - 1)
                def _():
                    start_idx(c0 + NBUF + b, b)

            return carry

        lax.fori_loop(0, n_grp, body, 0)
        for b in range(NBUF):
            wait_out(b)

    return gather_k


def _make_gather_add(B, T, D, CH, NBUF):
    b_per_w = B // NW
    n_ch = b_per_w // CH
    n_grp = n_ch // NBUF
    rpt = T // NS
    assert b_per_w % CH == 0 and B % (8 * NW) == 0 and CH % 8 == 0
    assert n_ch % NBUF == 0 and CH <= 128 and T % NS == 0 and rpt % 8 == 0

    @functools.partial(
        pl.kernel,
        mesh=_sc_mesh(),
        out_type=jax.ShapeDtypeStruct((B, D), jnp.float32),
        scratch_types=[
            *[pltpu.VMEM((CH,), jnp.int32) for _ in range(2 * NBUF)],
            *[pltpu.VMEM((CH, D), jnp.float32) for _ in range(2 * NBUF)],
            *[pltpu.SemaphoreType.DMA for _ in range(3 * NBUF)],
        ],
    )
    def gather_add_k(ta_hbm, ia_hbm, tb_hbm, ib_hbm, out_hbm, *rest):
        iabufs = rest[:NBUF]
        ibbufs = rest[NBUF:2 * NBUF]
        bufs = rest[2 * NBUF:3 * NBUF]
        bufsb = rest[3 * NBUF:4 * NBUF]
        isem = rest[4 * NBUF:5 * NBUF]
        gsem = rest[5 * NBUF:6 * NBUF]
        osem = rest[6 * NBUF:]
        sid = lax.axis_index("s")
        wid = sid * NC + lax.axis_index("c")
        base = wid * b_per_w

        def start_idx(c, b):
            pltpu.async_copy(ia_hbm.at[pl.ds(base + c * CH, CH)], iabufs[b],
                             isem[b])
            pltpu.async_copy(ib_hbm.at[pl.ds(base + c * CH, CH)], ibbufs[b],
                             isem[b])

        def wait_idx(b):
            pltpu.make_async_copy(ia_hbm.at[pl.ds(base, CH)], iabufs[b],
                                  isem[b]).wait()
            pltpu.make_async_copy(ib_hbm.at[pl.ds(base, CH)], ibbufs[b],
                                  isem[b]).wait()

        def start_ga(b):
            pltpu.async_copy(ta_hbm.at[iabufs[b]], bufs[b], gsem[b])

        def wait_g(b):
            pltpu.make_async_copy(ta_hbm.at[iabufs[b]], bufs[b],
                                  gsem[b]).wait()

        def start_gb(b):
            pltpu.async_copy(tb_hbm.at[ibbufs[b]], bufsb[b], gsem[b])

        def accum(b):
            def arow(r, carry):
                for j in range(D // LANES):
                    sl = pl.ds(j * LANES, LANES)
                    bufs[b][r, sl] = bufs[b][r, sl] + bufsb[b][r, sl]
                return carry

            lax.fori_loop(0, CH, arow, 0)

        def start_out(c, b):
            pltpu.async_copy(bufs[b], out_hbm.at[pl.ds(base + c * CH, CH)],
                             osem[b])

        def wait_out(b):
            pltpu.make_async_copy(bufs[b], out_hbm.at[pl.ds(base, CH)],
                                  osem[b]).wait()

        for b in range(NBUF):
            start_idx(b, b)

        def body(g, carry):
            c0 = g * NBUF
            for b in range(NBUF):
                wait_idx(b)

                @pl.when(g > 0)
                def _():
                    wait_out(b)

                start_ga(b)
                start_gb(b)
            for b in range(NBUF):
                wait_g(b)
                wait_g(b)
                accum(b)
                start_out(c0 + b, b)

                @pl.when(g < n_grp - 1)
                def _():
                    start_idx(c0 + NBUF + b, b)

            return carry

        lax.fori_loop(0, n_grp, body, 0)
        for b in range(NBUF):
            wait_out(b)

    return gather_add_k


def _make_scatter_add(B, T, D, CH, NBUF):
    b_per_w = B // NW
    n_ch = b_per_w // CH
    n_grp = n_ch // NBUF
    rpt = T // NS
    assert b_per_w % CH == 0 and T % NS == 0 and CH % 8 == 0 and rpt % 8 == 0
    assert n_ch % NBUF == 0 and CH <= 128

    @functools.partial(
        pl.kernel,
        mesh=_sc_mesh(),
        out_type=jax.ShapeDtypeStruct((NC, T, D), jnp.float32),
        scratch_types=[
            *[pltpu.VMEM((CH,), jnp.int32) for _ in range(NBUF)],
            *[pltpu.VMEM((CH, D), jnp.float32) for _ in range(NBUF)],
            pltpu.VMEM_SHARED((T, D), jnp.float32),
            *[pltpu.SemaphoreType.DMA for _ in range(2 * NBUF)],
        ],
    )
    def scatter_k(vals_hbm, dest_hbm, out_hbm, *rest):
        ibufs = rest[:NBUF]
        bufs = rest[NBUF:2 * NBUF]
        acc_sh = rest[2 * NBUF]
        lsem = rest[2 * NBUF + 1:3 * NBUF + 1]
        asem = rest[3 * NBUF + 1:]
        cid = lax.axis_index("c")
        sid = lax.axis_index("s")
        wid = sid * NC + cid
        base = wid * b_per_w

        zero = jnp.zeros((LANES,), jnp.float32)
        zch = min(CH, rpt)
        zfull, zrem = divmod(rpt, zch)
        assert zrem % 8 == 0

        def zrow(r, carry):
            for j in range(D // LANES):
                bufs[0][r, pl.ds(j * LANES, LANES)] = zero
            return carry

        def zcopy(k, carry):
            pltpu.sync_copy(
                bufs[0].at[pl.ds(0, zch)],
                acc_sh.at[pl.ds(sid * rpt + k * zch, zch)],
            )
            return carry

        def start_load(c, b):
            pltpu.async_copy(vals_hbm.at[pl.ds(base + c * CH, CH)], bufs[b],
                             lsem[b])
            pltpu.async_copy(dest_hbm.at[pl.ds(base + c * CH, CH)], ibufs[b],
                             lsem[b])

        def wait_load(b):
            pltpu.make_async_copy(vals_hbm.at[pl.ds(base, CH)], bufs[b],
                                  lsem[b]).wait()
            pltpu.make_async_copy(dest_hbm.at[pl.ds(base, CH)], ibufs[b],
                                  lsem[b]).wait()

        def start_add(b):
            pltpu.async_copy(bufs[b], acc_sh.at[ibufs[b]], asem[b], add=True)

        def wait_add(b):
            pltpu.make_async_copy(bufs[b], acc_sh.at[ibufs[b]],
                                  asem[b]).wait()

        for b in range(1, NBUF):
            start_load(b, b)
        lax.fori_loop(0, zch, zrow, 0)
        lax.fori_loop(0, zfull, zcopy, 0)
        if zrem:
            pltpu.sync_copy(
                bufs[0].at[pl.ds(0, zrem)],
                acc_sh.at[pl.ds(sid * rpt + zfull * zch, zrem)],
            )
        start_load(0, 0)
        plsc.subcore_barrier()

        def body(g, carry):
            c0 = g * NBUF
            for b in range(NBUF):
                wait_load(b)
                start_add(b)
            for b in range(NBUF):
                wait_add(b)

                @pl.when(g < n_grp - 1)
                def _():
                    start_load(c0 + NBUF + b, b)

            return carry

        lax.fori_loop(0, n_grp, body, 0)
        plsc.subcore_barrier()
        pltpu.sync_copy(
            acc_sh.at[pl.ds(sid * rpt, rpt)],
            out_hbm.at[cid, pl.ds(sid * rpt, rpt)],
        )

    return scatter_k


def _proj_body(x_ref, w_ref, o_ref):
    o_ref[...] = jnp.dot(x_ref[...], w_ref[...],
                         preferred_element_type=jnp.float32)


def _init_body(asrc_ref, e_ref, wb_ref, w_ref, h0_ref, hw_ref):
    h0 = jnp.maximum(
        asrc_ref[...]
        + jnp.dot(e_ref[...], wb_ref[...], preferred_element_type=jnp.float32),
        0.0,
    )
    h0_ref[...] = h0.astype(jnp.bfloat16)
    hw_ref[...] = h0 * w_ref[...]


def _addp_body(p_ref, o_ref):
    o_ref[...] = p_ref[0] + p_ref[1]


def _addp_neg_body(p_ref, o_ref):
    o_ref[...] = -(p_ref[0] + p_ref[1])


def _combine_body(d_ref, h0_ref, w_ref, whn_ref, hw_ref):
    h = jnp.maximum(
        jnp.dot(d_ref[...], whn_ref[...], preferred_element_type=jnp.float32)
        + h0_ref[...].astype(jnp.float32),
        0.0,
    )
    hw_ref[...] = h * w_ref[...]


def _combine_last_body(d_ref, h0_ref, whn_ref, h_ref):
    h_ref[...] = jnp.maximum(
        jnp.dot(d_ref[...], whn_ref[...], preferred_element_type=jnp.float32)
        + h0_ref[...].astype(jnp.float32),
        0.0,
    )


def _pool_body(n_mols):
    def body(b_ref, h_ref, o_ref):
        i = pl.program_id(0)
        br = b_ref.shape[2]
        mol_ids = lax.broadcasted_iota(jnp.int32, (n_mols, br), 0)
        onehot = (mol_ids == b_ref[0]).astype(jnp.float32)
        r = jnp.dot(onehot, h_ref[...], preferred_element_type=jnp.float32)

        @pl.when(i == 0)
        def _():
            o_ref[...] = r

        @pl.when(i > 0)
        def _():
            o_ref[...] += r

    return body


def _atom_body(v_ref, mf_ref, woa_ref, wob_ref, b_ref, o_ref):
    m = mf_ref[0] + mf_ref[1]
    o_ref[...] = jnp.maximum(
        jnp.dot(v_ref[...], woa_ref[...], preferred_element_type=jnp.float32)
        + jnp.dot(m, wob_ref[...], preferred_element_type=jnp.float32)
        + b_ref[...],
        0.0,
    )


def _row_spec(br, d):
    return pl.BlockSpec((br, d), lambda i: (i, 0))


def _full_spec(shape):
    return pl.BlockSpec(shape, lambda i: tuple(0 for _ in shape))


def kernel(V, E, edge_index, rev_edge_index, batch, weight, W_i, W_h, W_o, b_o):
    n_atoms, atom_dim = V.shape
    n_edges, bond_dim = E.shape
    hidden = W_h.shape[0]

    src = edge_index[0]
    dest = edge_index[1]
    w2 = weight[:, None]
    wa = W_i[:atom_dim]
    wb = W_i[atom_dim:]
    woa = W_o[:atom_dim]
    wob = W_o[atom_dim:]
    b2 = b_o[None, :]

    BR_E = 8000
    BR_A = 2000
    CH = 80
    CHS = 40
    NBUF = 5

    t_acc = 10112
    gather = _make_gather(n_edges, hidden, CH, NBUF)
    scatter_edges = _make_scatter_add(n_edges, t_acc, hidden, CHS, NBUF)

    A = pl.pallas_call(
        _proj_body,
        grid=(n_atoms // BR_A,),
        in_specs=[_row_spec(BR_A, atom_dim), _full_spec((atom_dim, hidden))],
        out_specs=_row_spec(BR_A, hidden),
        out_shape=jax.ShapeDtypeStruct((n_atoms, hidden), jnp.float32),
    )(V, wa)

    a_src = gather(A, src)

    h0, hw = pl.pallas_call(
        _init_body,
        grid=(n_edges // BR_E,),
        in_specs=[
            _row_spec(BR_E, hidden),
            _row_spec(BR_E, bond_dim),
            _full_spec((bond_dim, hidden)),
            _row_spec(BR_E, 1),
        ],
        out_specs=[_row_spec(BR_E, hidden), _row_spec(BR_E, hidden)],
        out_shape=[
            jax.ShapeDtypeStruct((n_edges, hidden), jnp.bfloat16),
            jax.ShapeDtypeStruct((n_edges, hidden), jnp.float32),
        ],
    )(a_src, E, wb, w2)

    add_partials_neg = pl.pallas_call(
        _addp_neg_body,
        grid=(t_acc // 1264,),
        in_specs=[pl.BlockSpec((NC, 1264, hidden), lambda i: (0, i, 0))],
        out_specs=_row_spec(1264, hidden),
        out_shape=jax.ShapeDtypeStruct((t_acc, hidden), jnp.float32),
    )

    combine = pl.pallas_call(
        _combine_body,
        grid=(n_edges // BR_E,),
        in_specs=[
            _row_spec(BR_E, hidden),
            _row_spec(BR_E, hidden),
            _row_spec(BR_E, 1),
            _full_spec((hidden, hidden)),
        ],
        out_specs=_row_spec(BR_E, hidden),
        out_shape=jax.ShapeDtypeStruct((n_edges, hidden), jnp.float32),
    )

    combine_last = pl.pallas_call(
        _combine_last_body,
        grid=(n_edges // BR_E,),
        in_specs=[
            _row_spec(BR_E, hidden),
            _row_spec(BR_E, hidden),
            _full_spec((hidden, hidden)),
        ],
        out_specs=_row_spec(BR_E, hidden),
        out_shape=jax.ShapeDtypeStruct((n_edges, hidden), jnp.float32),
    )

    gather_add = _make_gather_add(n_edges, t_acc, hidden, CH, NBUF)
    wh_neg = -W_h

    for it in range(3):
        s_neg = add_partials_neg(scatter_edges(hw, dest))
        d = gather_add(s_neg, src, hw, rev_edge_index)
        if it < 2:
            hw = combine(d, h0, w2, wh_neg)
        else:
            h = combine_last(d, h0, wh_neg)

    mf = scatter_edges(h, dest)

    h_atom = pl.pallas_call(
        _atom_body,
        grid=(n_atoms // BR_A,),
        in_specs=[
            _row_spec(BR_A, atom_dim),
            pl.BlockSpec((NC, BR_A, hidden), lambda i: (0, i, 0)),
            _full_spec((atom_dim, hidden)),
            _full_spec((hidden, hidden)),
            pl.BlockSpec((1, hidden), lambda i: (0, 0)),
        ],
        out_specs=_row_spec(BR_A, hidden),
        out_shape=jax.ShapeDtypeStruct((n_atoms, hidden), jnp.float32),
    )(V, mf, woa, wob, b2)

    n_mols = 256
    b2d = batch.reshape(n_atoms // BR_A, 1, BR_A)
    mol_vecs = pl.pallas_call(
        _pool_body(n_mols),
        grid=(n_atoms // BR_A,),
        in_specs=[
            pl.BlockSpec((1, 1, BR_A), lambda i: (i, 0, 0)),
            _row_spec(BR_A, hidden),
        ],
        out_specs=pl.BlockSpec((n_mols, hidden), lambda i: (0, 0)),
        out_shape=jax.ShapeDtypeStruct((n_mols, hidden), jnp.float32),
    )(b2d, h_atom)

    return (h_atom, batch, mol_vecs, h)
